# Initial kernel scaffold; baseline (speedup 1.0000x reference)
import jax
import jax.numpy as jnp
from jax.experimental import pallas as pl


def kernel(fmess, bgraph, Wz_w, Wz_b, Wr_w, Ur_w, Ur_b, Wh_w, Wh_b):
    raise NotImplementedError("write your pallas kernel here")


# speedup vs baseline: 3.2980x; 3.2980x over previous
import functools

import jax
import jax.numpy as jnp
from jax import lax
from jax.experimental import pallas as pl
from jax.experimental.pallas import tpu as pltpu
from jax.experimental.pallas import tpu_sc as plsc

E = 160000
K = 6
D = 128

_NC = 2
_NS = 16
_NW = _NC * _NS
_CH = 120
_B = E * K
_PER_W = _B // _NW
_NCHUNK = _PER_W // _CH

_RP = 640
_RU = 320


def _sigmoid(x):
    return 1.0 / (1.0 + jnp.exp(-x))


def _sc_gather(table, idx):
    mesh = plsc.VectorSubcoreMesh(core_axis_name="c", subcore_axis_name="s")

    @functools.partial(
        pl.kernel,
        mesh=mesh,
        out_type=jax.ShapeDtypeStruct((_B, 2 * D), jnp.float32),
        scratch_types=[
            pltpu.VMEM((_CH,), jnp.int32),
            pltpu.VMEM((_CH, 2 * D), jnp.float32),
            pltpu.SemaphoreType.DMA,
        ],
    )
    def gather_kernel(table_hbm, idx_hbm, out_hbm, idx_v, rows_v, sem):
        wid = lax.axis_index("s") * _NC + lax.axis_index("c")
        base = wid * _PER_W

        def body(c, carry):
            off = pl.multiple_of(base + c * _CH, 8)
            pltpu.sync_copy(idx_hbm.at[pl.ds(off, _CH)], idx_v)
            pltpu.async_copy(table_hbm.at[idx_v], rows_v, sem).wait()
            pltpu.sync_copy(rows_v, out_hbm.at[pl.ds(off, _CH)])
            return carry

        lax.fori_loop(0, _NCHUNK, body, 0)

    return gather_kernel(table, idx)


def _precompute(fmess, w_x, b_x, ur_t):
    grid = E // _RP

    def body(x_ref, w_ref, b_ref, u_ref, xzrh_ref, hcat_ref):
        x = x_ref[...]
        xzrh = jnp.dot(x, w_ref[...], preferred_element_type=jnp.float32) + b_ref[...]
        xzrh_ref[...] = xzrh
        h1 = _sigmoid(xzrh[:, :D]) * jnp.tanh(xzrh[:, 2 * D:])
        rows = lax.broadcasted_iota(jnp.int32, (_RP, 1), 0)
        keep = jnp.where((rows == 0) & (pl.program_id(0) == 0), 0.0, 1.0)
        h1 = h1 * keep
        hu1 = jnp.dot(h1, u_ref[...], preferred_element_type=jnp.float32)
        hcat_ref[...] = jnp.concatenate([h1, hu1], axis=1)

    return pl.pallas_call(
        body,
        grid=(grid,),
        in_specs=[
            pl.BlockSpec((_RP, D), lambda i: (i, 0)),
            pl.BlockSpec((D, 3 * D), lambda i: (0, 0)),
            pl.BlockSpec((1, 3 * D), lambda i: (0, 0)),
            pl.BlockSpec((D, D), lambda i: (0, 0)),
        ],
        out_specs=[
            pl.BlockSpec((_RP, 3 * D), lambda i: (i, 0)),
            pl.BlockSpec((_RP, 2 * D), lambda i: (i, 0)),
        ],
        out_shape=[
            jax.ShapeDtypeStruct((E, 3 * D), jnp.float32),
            jax.ShapeDtypeStruct((E, 2 * D), jnp.float32),
        ],
    )(fmess, w_x, b_x, ur_t)


def _update(gathered, xzrh, wz_h_t, wh_h_t, ur_t, ur_b, last):
    grid = E // _RU
    out_d = D if last else 2 * D

    def body(g_ref, x_ref, wz_ref, wh_ref, u_ref, ub_ref, out_ref):
        x = x_ref[...]
        xz = x[:, :D]
        xr = x[:, D:2 * D]
        xh = x[:, 2 * D:]
        ub = ub_ref[...]
        sum_h = jnp.zeros((_RU, D), jnp.float32)
        sum_g = jnp.zeros((_RU, D), jnp.float32)
        for k in range(K):
            hk = g_ref[k, :, :D]
            huk = g_ref[k, :, D:]
            sum_h = sum_h + hk
            rk = _sigmoid(xr + huk + ub)
            sum_g = sum_g + rk * hk
        z = _sigmoid(xz + jnp.dot(sum_h, wz_ref[...], preferred_element_type=jnp.float32))
        pre = jnp.tanh(xh + jnp.dot(sum_g, wh_ref[...], preferred_element_type=jnp.float32))
        h = (1.0 - z) * sum_h + z * pre
        rows = lax.broadcasted_iota(jnp.int32, (_RU, 1), 0)
        keep = jnp.where((rows == 0) & (pl.program_id(0) == 0), 0.0, 1.0)
        h = h * keep
        if last:
            out_ref[...] = h
        else:
            hu = jnp.dot(h, u_ref[...], preferred_element_type=jnp.float32)
            out_ref[...] = jnp.concatenate([h, hu], axis=1)

    return pl.pallas_call(
        body,
        grid=(grid,),
        in_specs=[
            pl.BlockSpec((K, _RU, 2 * D), lambda i: (0, i, 0)),
            pl.BlockSpec((_RU, 3 * D), lambda i: (i, 0)),
            pl.BlockSpec((D, D), lambda i: (0, 0)),
            pl.BlockSpec((D, D), lambda i: (0, 0)),
            pl.BlockSpec((D, D), lambda i: (0, 0)),
            pl.BlockSpec((1, D), lambda i: (0, 0)),
        ],
        out_specs=pl.BlockSpec((_RU, out_d), lambda i: (i, 0)),
        out_shape=jax.ShapeDtypeStruct((E, out_d), jnp.float32),
    )(gathered, xzrh, wz_h_t, wh_h_t, ur_t, ur_b)


def kernel(fmess, bgraph, Wz_w, Wz_b, Wr_w, Ur_w, Ur_b, Wh_w, Wh_b):
    bg = bgraph.astype(jnp.int32)
    idx = bg.T.reshape(_B)
    w_x = jnp.concatenate([Wz_w[:, :D].T, Wr_w.T, Wh_w[:, :D].T], axis=1)
    b_x = jnp.concatenate([Wz_b, jnp.zeros((D,), jnp.float32), Wh_b]).reshape(1, 3 * D)
    ur_t = Ur_w.T
    wz_h_t = Wz_w[:, D:].T
    wh_h_t = Wh_w[:, D:].T
    ur_b = Ur_b.reshape(1, D)

    xzrh, hcat = _precompute(fmess, w_x, b_x, ur_t)
    h = None
    for d in range(2):
        gathered = _sc_gather(hcat, idx).reshape(K, E, 2 * D)
        last = d == 1
        out = _update(gathered, xzrh, wz_h_t, wh_h_t, ur_t, ur_b, last)
        if last:
            h = out
        else:
            hcat = out
    return h


# speedup vs baseline: 4.3219x; 1.3105x over previous
import functools

import jax
import jax.numpy as jnp
from jax import lax
from jax.experimental import pallas as pl
from jax.experimental.pallas import tpu as pltpu
from jax.experimental.pallas import tpu_sc as plsc

E = 160000
K = 6
D = 128

_NC = 2
_NS = 16
_NW = _NC * _NS
_CH = 120
_B = E * K
_PER_W = _B // _NW
_NCHUNK = _PER_W // _CH

_RP = 640
_RU = 320


def _sigmoid(x):
    return 1.0 / (1.0 + jnp.exp(-x))


def _pack(h, hu):
    hb = lax.bitcast_convert_type(h, jnp.int32) + 0x8000
    ub = lax.bitcast_convert_type(hu, jnp.int32) + 0x8000
    hi = jnp.bitwise_and(hb, jnp.int32(-65536))
    lo = lax.shift_right_logical(ub, 16)
    return jnp.bitwise_or(hi, lo)


def _unpack(w):
    h = lax.bitcast_convert_type(jnp.bitwise_and(w, jnp.int32(-65536)), jnp.float32)
    hu = lax.bitcast_convert_type(lax.shift_left(w, 16), jnp.float32)
    return h, hu


def _sc_gather(table, idx):
    mesh = plsc.VectorSubcoreMesh(core_axis_name="c", subcore_axis_name="s")

    @functools.partial(
        pl.kernel,
        mesh=mesh,
        out_type=jax.ShapeDtypeStruct((_B, D), jnp.int32),
        scratch_types=[
            pltpu.VMEM((_CH,), jnp.int32),
            pltpu.VMEM((_CH, D), jnp.int32),
            pltpu.SemaphoreType.DMA,
        ],
    )
    def gather_kernel(table_hbm, idx_hbm, out_hbm, idx_v, rows_v, sem):
        wid = lax.axis_index("s") * _NC + lax.axis_index("c")
        base = wid * _PER_W

        def body(c, carry):
            off = pl.multiple_of(base + c * _CH, 8)
            pltpu.sync_copy(idx_hbm.at[pl.ds(off, _CH)], idx_v)
            pltpu.async_copy(table_hbm.at[idx_v], rows_v, sem).wait()
            pltpu.sync_copy(rows_v, out_hbm.at[pl.ds(off, _CH)])
            return carry

        lax.fori_loop(0, _NCHUNK, body, 0)

    return gather_kernel(table, idx)


def _precompute(fmess, w_x, b_x, ur_t):
    grid = E // _RP

    def body(x_ref, w_ref, b_ref, u_ref, xzrh_ref, hcat_ref):
        x = x_ref[...]
        xzrh = jnp.dot(x, w_ref[...], preferred_element_type=jnp.float32) + b_ref[...]
        xzrh_ref[...] = xzrh
        h1 = _sigmoid(xzrh[:, :D]) * jnp.tanh(xzrh[:, 2 * D:])
        rows = lax.broadcasted_iota(jnp.int32, (_RP, 1), 0)
        keep = jnp.where((rows == 0) & (pl.program_id(0) == 0), 0.0, 1.0)
        h1 = h1 * keep
        hu1 = jnp.dot(h1, u_ref[...], preferred_element_type=jnp.float32)
        hcat_ref[...] = _pack(h1, hu1)

    return pl.pallas_call(
        body,
        grid=(grid,),
        in_specs=[
            pl.BlockSpec((_RP, D), lambda i: (i, 0)),
            pl.BlockSpec((D, 3 * D), lambda i: (0, 0)),
            pl.BlockSpec((1, 3 * D), lambda i: (0, 0)),
            pl.BlockSpec((D, D), lambda i: (0, 0)),
        ],
        out_specs=[
            pl.BlockSpec((_RP, 3 * D), lambda i: (i, 0)),
            pl.BlockSpec((_RP, D), lambda i: (i, 0)),
        ],
        out_shape=[
            jax.ShapeDtypeStruct((E, 3 * D), jnp.float32),
            jax.ShapeDtypeStruct((E, D), jnp.int32),
        ],
    )(fmess, w_x, b_x, ur_t)


def _update(gathered, xzrh, wz_h_t, wh_h_t, ur_t, ur_b, last):
    grid = E // _RU

    def body(g_ref, x_ref, wz_ref, wh_ref, u_ref, ub_ref, out_ref):
        x = x_ref[...]
        xz = x[:, :D]
        xr = x[:, D:2 * D]
        xh = x[:, 2 * D:]
        ub = ub_ref[...]
        sum_h = jnp.zeros((_RU, D), jnp.float32)
        sum_g = jnp.zeros((_RU, D), jnp.float32)
        for k in range(K):
            hk, huk = _unpack(g_ref[k, :, :])
            sum_h = sum_h + hk
            rk = _sigmoid(xr + huk + ub)
            sum_g = sum_g + rk * hk
        z = _sigmoid(xz + jnp.dot(sum_h, wz_ref[...], preferred_element_type=jnp.float32))
        pre = jnp.tanh(xh + jnp.dot(sum_g, wh_ref[...], preferred_element_type=jnp.float32))
        h = (1.0 - z) * sum_h + z * pre
        rows = lax.broadcasted_iota(jnp.int32, (_RU, 1), 0)
        keep = jnp.where((rows == 0) & (pl.program_id(0) == 0), 0.0, 1.0)
        h = h * keep
        if last:
            out_ref[...] = h
        else:
            hu = jnp.dot(h, u_ref[...], preferred_element_type=jnp.float32)
            out_ref[...] = _pack(h, hu)

    return pl.pallas_call(
        body,
        grid=(grid,),
        in_specs=[
            pl.BlockSpec((K, _RU, D), lambda i: (0, i, 0)),
            pl.BlockSpec((_RU, 3 * D), lambda i: (i, 0)),
            pl.BlockSpec((D, D), lambda i: (0, 0)),
            pl.BlockSpec((D, D), lambda i: (0, 0)),
            pl.BlockSpec((D, D), lambda i: (0, 0)),
            pl.BlockSpec((1, D), lambda i: (0, 0)),
        ],
        out_specs=pl.BlockSpec((_RU, D), lambda i: (i, 0)),
        out_shape=jax.ShapeDtypeStruct(
            (E, D), jnp.float32 if last else jnp.int32),
    )(gathered, xzrh, wz_h_t, wh_h_t, ur_t, ur_b)


def kernel(fmess, bgraph, Wz_w, Wz_b, Wr_w, Ur_w, Ur_b, Wh_w, Wh_b):
    bg = bgraph.astype(jnp.int32)
    idx = bg.T.reshape(_B)
    w_x = jnp.concatenate([Wz_w[:, :D].T, Wr_w.T, Wh_w[:, :D].T], axis=1)
    b_x = jnp.concatenate([Wz_b, jnp.zeros((D,), jnp.float32), Wh_b]).reshape(1, 3 * D)
    ur_t = Ur_w.T
    wz_h_t = Wz_w[:, D:].T
    wh_h_t = Wh_w[:, D:].T
    ur_b = Ur_b.reshape(1, D)

    xzrh, hcat = _precompute(fmess, w_x, b_x, ur_t)
    h = None
    for d in range(2):
        gathered = _sc_gather(hcat, idx).reshape(K, E, D)
        last = d == 1
        out = _update(gathered, xzrh, wz_h_t, wh_h_t, ur_t, ur_b, last)
        if last:
            h = out
        else:
            hcat = out
    return h


# speedup vs baseline: 5.6414x; 1.3053x over previous
import functools

import jax
import jax.numpy as jnp
from jax import lax
from jax.experimental import pallas as pl
from jax.experimental.pallas import tpu as pltpu
from jax.experimental.pallas import tpu_sc as plsc

E = 160000
K = 6
D = 128

_NC = 2
_NS = 16
_NW = _NC * _NS
_CH = 120
_B = E * K
_PER_W = _B // _NW
_NCHUNK = _PER_W // _CH

_RP = 640
_RU = 320


def _sigmoid(x):
    return 1.0 / (1.0 + jnp.exp(-x))


def _pack(h, hu):
    hb = lax.bitcast_convert_type(h, jnp.int32) + 0x8000
    ub = lax.bitcast_convert_type(hu, jnp.int32) + 0x8000
    hi = jnp.bitwise_and(hb, jnp.int32(-65536))
    lo = lax.shift_right_logical(ub, 16)
    return jnp.bitwise_or(hi, lo)


def _unpack(w):
    h = lax.bitcast_convert_type(jnp.bitwise_and(w, jnp.int32(-65536)), jnp.float32)
    hu = lax.bitcast_convert_type(lax.shift_left(w, 16), jnp.float32)
    return h, hu


def _sc_gather(table, idx):
    mesh = plsc.VectorSubcoreMesh(core_axis_name="c", subcore_axis_name="s")

    @functools.partial(
        pl.kernel,
        mesh=mesh,
        out_type=jax.ShapeDtypeStruct((_B, D), jnp.int32),
        scratch_types=[
            pltpu.VMEM((_PER_W,), jnp.int32),
            pltpu.VMEM((_CH, D), jnp.int32),
            pltpu.VMEM((_CH, D), jnp.int32),
            pltpu.SemaphoreType.DMA,
            pltpu.SemaphoreType.DMA,
            pltpu.SemaphoreType.DMA,
            pltpu.SemaphoreType.DMA,
        ],
    )
    def gather_kernel(table_hbm, idx_hbm, out_hbm, idx_all,
                      b0, b1, g0, g1, s0, s1):
        wid = lax.axis_index("s") * _NC + lax.axis_index("c")
        base = wid * _PER_W
        rows = (b0, b1)
        gsem = (g0, g1)
        ssem = (s0, s1)

        pltpu.sync_copy(idx_hbm.at[pl.ds(base, _PER_W)], idx_all)

        def fire_g(c, b):
            pltpu.async_copy(
                table_hbm.at[idx_all.at[pl.ds(c * _CH, _CH)]], rows[b], gsem[b])

        def wait_g(c, b):
            pltpu.make_async_copy(
                table_hbm.at[idx_all.at[pl.ds(c * _CH, _CH)]], rows[b],
                gsem[b]).wait()

        def fire_s(c, b):
            pltpu.async_copy(
                rows[b], out_hbm.at[pl.ds(base + c * _CH, _CH)], ssem[b])

        def wait_s(c, b):
            pltpu.make_async_copy(
                rows[b], out_hbm.at[pl.ds(base + c * _CH, _CH)], ssem[b]).wait()

        fire_g(0, 0)

        def body(i, carry):
            c = i * 2

            @pl.when(i > 0)
            def _():
                wait_s(c - 1, 1)

            fire_g(c + 1, 1)
            wait_g(c, 0)
            fire_s(c, 0)

            wait_s(c, 0)

            @pl.when(i < _NCHUNK // 2 - 1)
            def _():
                fire_g(c + 2, 0)

            wait_g(c + 1, 1)
            fire_s(c + 1, 1)
            return carry

        lax.fori_loop(0, _NCHUNK // 2, body, 0)
        wait_s(_NCHUNK - 1, 1)

    return gather_kernel(table, idx)


def _precompute(fmess, w_x, b_x, ur_t):
    grid = E // _RP

    def body(x_ref, w_ref, b_ref, u_ref, xzrh_ref, hcat_ref):
        x = x_ref[...]
        xzrh = jnp.dot(x, w_ref[...], preferred_element_type=jnp.float32) + b_ref[...]
        xzrh_ref[...] = xzrh
        h1 = _sigmoid(xzrh[:, :D]) * jnp.tanh(xzrh[:, 2 * D:])
        rows = lax.broadcasted_iota(jnp.int32, (_RP, 1), 0)
        keep = jnp.where((rows == 0) & (pl.program_id(0) == 0), 0.0, 1.0)
        h1 = h1 * keep
        hu1 = jnp.dot(h1, u_ref[...], preferred_element_type=jnp.float32)
        hcat_ref[...] = _pack(h1, hu1)

    return pl.pallas_call(
        body,
        grid=(grid,),
        in_specs=[
            pl.BlockSpec((_RP, D), lambda i: (i, 0)),
            pl.BlockSpec((D, 3 * D), lambda i: (0, 0)),
            pl.BlockSpec((1, 3 * D), lambda i: (0, 0)),
            pl.BlockSpec((D, D), lambda i: (0, 0)),
        ],
        out_specs=[
            pl.BlockSpec((_RP, 3 * D), lambda i: (i, 0)),
            pl.BlockSpec((_RP, D), lambda i: (i, 0)),
        ],
        out_shape=[
            jax.ShapeDtypeStruct((E, 3 * D), jnp.float32),
            jax.ShapeDtypeStruct((E, D), jnp.int32),
        ],
    )(fmess, w_x, b_x, ur_t)


def _update(gathered, xzrh, wz_h_t, wh_h_t, ur_t, ur_b, last):
    grid = E // _RU

    def body(g_ref, x_ref, wz_ref, wh_ref, u_ref, ub_ref, out_ref):
        x = x_ref[...]
        xz = x[:, :D]
        xr = x[:, D:2 * D]
        xh = x[:, 2 * D:]
        ub = ub_ref[...]
        sum_h = jnp.zeros((_RU, D), jnp.float32)
        sum_g = jnp.zeros((_RU, D), jnp.float32)
        for k in range(K):
            hk, huk = _unpack(g_ref[k, :, :])
            sum_h = sum_h + hk
            rk = _sigmoid(xr + huk + ub)
            sum_g = sum_g + rk * hk
        z = _sigmoid(xz + jnp.dot(sum_h, wz_ref[...], preferred_element_type=jnp.float32))
        pre = jnp.tanh(xh + jnp.dot(sum_g, wh_ref[...], preferred_element_type=jnp.float32))
        h = (1.0 - z) * sum_h + z * pre
        rows = lax.broadcasted_iota(jnp.int32, (_RU, 1), 0)
        keep = jnp.where((rows == 0) & (pl.program_id(0) == 0), 0.0, 1.0)
        h = h * keep
        if last:
            out_ref[...] = h
        else:
            hu = jnp.dot(h, u_ref[...], preferred_element_type=jnp.float32)
            out_ref[...] = _pack(h, hu)

    return pl.pallas_call(
        body,
        grid=(grid,),
        in_specs=[
            pl.BlockSpec((K, _RU, D), lambda i: (0, i, 0)),
            pl.BlockSpec((_RU, 3 * D), lambda i: (i, 0)),
            pl.BlockSpec((D, D), lambda i: (0, 0)),
            pl.BlockSpec((D, D), lambda i: (0, 0)),
            pl.BlockSpec((D, D), lambda i: (0, 0)),
            pl.BlockSpec((1, D), lambda i: (0, 0)),
        ],
        out_specs=pl.BlockSpec((_RU, D), lambda i: (i, 0)),
        out_shape=jax.ShapeDtypeStruct(
            (E, D), jnp.float32 if last else jnp.int32),
    )(gathered, xzrh, wz_h_t, wh_h_t, ur_t, ur_b)


def kernel(fmess, bgraph, Wz_w, Wz_b, Wr_w, Ur_w, Ur_b, Wh_w, Wh_b):
    bg = bgraph.astype(jnp.int32)
    idx = bg.T.reshape(_B)
    w_x = jnp.concatenate([Wz_w[:, :D].T, Wr_w.T, Wh_w[:, :D].T], axis=1)
    b_x = jnp.concatenate([Wz_b, jnp.zeros((D,), jnp.float32), Wh_b]).reshape(1, 3 * D)
    ur_t = Ur_w.T
    wz_h_t = Wz_w[:, D:].T
    wh_h_t = Wh_w[:, D:].T
    ur_b = Ur_b.reshape(1, D)

    xzrh, hcat = _precompute(fmess, w_x, b_x, ur_t)
    h = None
    for d in range(2):
        gathered = _sc_gather(hcat, idx).reshape(K, E, D)
        last = d == 1
        out = _update(gathered, xzrh, wz_h_t, wh_h_t, ur_t, ur_b, last)
        if last:
            h = out
        else:
            hcat = out
    return h


# speedup vs baseline: 7.3620x; 1.3050x over previous
import functools

import jax
import jax.numpy as jnp
from jax import lax
from jax.experimental import pallas as pl
from jax.experimental.pallas import tpu as pltpu
from jax.experimental.pallas import tpu_sc as plsc

E = 160000
K = 6
D = 128

_NC = 2
_NS = 16
_NW = _NC * _NS
_CH = 120
_B = E * K
_PER_W = _B // _NW
_NCHUNK = _PER_W // _CH

_RP = 1600
_RU = 800


def _sigmoid(x):
    return 1.0 / (1.0 + jnp.exp(-x))


def _pack(h, hu):
    hb = lax.bitcast_convert_type(h, jnp.int32) + 0x8000
    ub = lax.bitcast_convert_type(hu, jnp.int32) + 0x8000
    hi = jnp.bitwise_and(hb, jnp.int32(-65536))
    lo = lax.shift_right_logical(ub, 16)
    return jnp.bitwise_or(hi, lo)


def _unpack(w):
    h = lax.bitcast_convert_type(jnp.bitwise_and(w, jnp.int32(-65536)), jnp.float32)
    hu = lax.bitcast_convert_type(lax.shift_left(w, 16), jnp.float32)
    return h, hu


def _sc_gather(table, idx):
    mesh = plsc.VectorSubcoreMesh(core_axis_name="c", subcore_axis_name="s")

    @functools.partial(
        pl.kernel,
        mesh=mesh,
        out_type=jax.ShapeDtypeStruct((_B, D), jnp.int32),
        scratch_types=[
            pltpu.VMEM((_PER_W,), jnp.int32),
            pltpu.VMEM((_CH, D), jnp.int32),
            pltpu.VMEM((_CH, D), jnp.int32),
            pltpu.SemaphoreType.DMA,
            pltpu.SemaphoreType.DMA,
            pltpu.SemaphoreType.DMA,
            pltpu.SemaphoreType.DMA,
        ],
    )
    def gather_kernel(table_hbm, idx_hbm, out_hbm, idx_all,
                      b0, b1, g0, g1, s0, s1):
        wid = lax.axis_index("s") * _NC + lax.axis_index("c")
        base = wid * _PER_W
        rows = (b0, b1)
        gsem = (g0, g1)
        ssem = (s0, s1)

        pltpu.sync_copy(idx_hbm.at[pl.ds(base, _PER_W)], idx_all)

        def fire_g(c, b):
            pltpu.async_copy(
                table_hbm.at[idx_all.at[pl.ds(c * _CH, _CH)]], rows[b], gsem[b])

        def wait_g(c, b):
            pltpu.make_async_copy(
                table_hbm.at[idx_all.at[pl.ds(c * _CH, _CH)]], rows[b],
                gsem[b]).wait()

        def fire_s(c, b):
            pltpu.async_copy(
                rows[b], out_hbm.at[pl.ds(base + c * _CH, _CH)], ssem[b])

        def wait_s(c, b):
            pltpu.make_async_copy(
                rows[b], out_hbm.at[pl.ds(base + c * _CH, _CH)], ssem[b]).wait()

        fire_g(0, 0)

        def body(i, carry):
            c = i * 2

            @pl.when(i > 0)
            def _():
                wait_s(c - 1, 1)

            fire_g(c + 1, 1)
            wait_g(c, 0)
            fire_s(c, 0)

            wait_s(c, 0)

            @pl.when(i < _NCHUNK // 2 - 1)
            def _():
                fire_g(c + 2, 0)

            wait_g(c + 1, 1)
            fire_s(c + 1, 1)
            return carry

        lax.fori_loop(0, _NCHUNK // 2, body, 0)
        wait_s(_NCHUNK - 1, 1)

    return gather_kernel(table, idx)


def _precompute(fmess, w_x, b_x, ur_t):
    grid = E // _RP

    def body(x_ref, w_ref, b_ref, u_ref, xzrh_ref, hcat_ref):
        x = x_ref[...]
        xzrh = jnp.dot(x, w_ref[...], preferred_element_type=jnp.float32) + b_ref[...]
        xzrh_ref[...] = xzrh.astype(jnp.bfloat16)
        h1 = _sigmoid(xzrh[:, :D]) * jnp.tanh(xzrh[:, 2 * D:])
        rows = lax.broadcasted_iota(jnp.int32, (_RP, 1), 0)
        keep = jnp.where((rows == 0) & (pl.program_id(0) == 0), 0.0, 1.0)
        h1 = h1 * keep
        hu1 = jnp.dot(h1, u_ref[...], preferred_element_type=jnp.float32)
        hcat_ref[...] = _pack(h1, hu1)

    return pl.pallas_call(
        body,
        grid=(grid,),
        in_specs=[
            pl.BlockSpec((_RP, D), lambda i: (i, 0)),
            pl.BlockSpec((D, 3 * D), lambda i: (0, 0)),
            pl.BlockSpec((1, 3 * D), lambda i: (0, 0)),
            pl.BlockSpec((D, D), lambda i: (0, 0)),
        ],
        out_specs=[
            pl.BlockSpec((_RP, 3 * D), lambda i: (i, 0)),
            pl.BlockSpec((_RP, D), lambda i: (i, 0)),
        ],
        out_shape=[
            jax.ShapeDtypeStruct((E, 3 * D), jnp.bfloat16),
            jax.ShapeDtypeStruct((E, D), jnp.int32),
        ],
    )(fmess, w_x, b_x, ur_t)


def _update(gathered, xzrh, wz_h_t, wh_h_t, ur_t, ur_b, last):
    grid = E // _RU

    def body(g_ref, x_ref, wz_ref, wh_ref, u_ref, ub_ref, out_ref):
        x = x_ref[...].astype(jnp.float32)
        xz = x[:, :D]
        xr = x[:, D:2 * D]
        xh = x[:, 2 * D:]
        ub = ub_ref[...]
        sum_h = jnp.zeros((_RU, D), jnp.float32)
        sum_g = jnp.zeros((_RU, D), jnp.float32)
        for k in range(K):
            hk, huk = _unpack(g_ref[k, :, :])
            sum_h = sum_h + hk
            rk = _sigmoid(xr + huk + ub)
            sum_g = sum_g + rk * hk
        z = _sigmoid(xz + jnp.dot(sum_h, wz_ref[...], preferred_element_type=jnp.float32))
        pre = jnp.tanh(xh + jnp.dot(sum_g, wh_ref[...], preferred_element_type=jnp.float32))
        h = (1.0 - z) * sum_h + z * pre
        rows = lax.broadcasted_iota(jnp.int32, (_RU, 1), 0)
        keep = jnp.where((rows == 0) & (pl.program_id(0) == 0), 0.0, 1.0)
        h = h * keep
        if last:
            out_ref[...] = h
        else:
            hu = jnp.dot(h, u_ref[...], preferred_element_type=jnp.float32)
            out_ref[...] = _pack(h, hu)

    return pl.pallas_call(
        body,
        grid=(grid,),
        in_specs=[
            pl.BlockSpec((K, _RU, D), lambda i: (0, i, 0)),
            pl.BlockSpec((_RU, 3 * D), lambda i: (i, 0)),
            pl.BlockSpec((D, D), lambda i: (0, 0)),
            pl.BlockSpec((D, D), lambda i: (0, 0)),
            pl.BlockSpec((D, D), lambda i: (0, 0)),
            pl.BlockSpec((1, D), lambda i: (0, 0)),
        ],
        out_specs=pl.BlockSpec((_RU, D), lambda i: (i, 0)),
        out_shape=jax.ShapeDtypeStruct(
            (E, D), jnp.float32 if last else jnp.int32),
    )(gathered, xzrh, wz_h_t, wh_h_t, ur_t, ur_b)


def kernel(fmess, bgraph, Wz_w, Wz_b, Wr_w, Ur_w, Ur_b, Wh_w, Wh_b):
    bg = bgraph.astype(jnp.int32)
    idx = bg.T.reshape(_B)
    w_x = jnp.concatenate([Wz_w[:, :D].T, Wr_w.T, Wh_w[:, :D].T], axis=1)
    b_x = jnp.concatenate([Wz_b, jnp.zeros((D,), jnp.float32), Wh_b]).reshape(1, 3 * D)
    ur_t = Ur_w.T
    wz_h_t = Wz_w[:, D:].T
    wh_h_t = Wh_w[:, D:].T
    ur_b = Ur_b.reshape(1, D)

    xzrh, hcat = _precompute(fmess, w_x, b_x, ur_t)
    h = None
    for d in range(2):
        gathered = _sc_gather(hcat, idx).reshape(K, E, D)
        last = d == 1
        out = _update(gathered, xzrh, wz_h_t, wh_h_t, ur_t, ur_b, last)
        if last:
            h = out
        else:
            hcat = out
    return h


# speedup vs baseline: 7.4998x; 1.0187x over previous
import functools

import jax
import jax.numpy as jnp
from jax import lax
from jax.experimental import pallas as pl
from jax.experimental.pallas import tpu as pltpu
from jax.experimental.pallas import tpu_sc as plsc

E = 160000
K = 6
D = 128

_NC = 2
_NS = 16
_NW = _NC * _NS
_CH = 120
_B = E * K
_PER_W = _B // _NW
_NCHUNK = _PER_W // _CH

_RP = 1600
_RU = 800


def _sigmoid(x):
    return 0.5 * jnp.tanh(0.5 * x) + 0.5


def _pack(h, hu):
    hb = lax.bitcast_convert_type(h, jnp.int32) + 0x8000
    ub = lax.bitcast_convert_type(hu, jnp.int32) + 0x8000
    hi = jnp.bitwise_and(hb, jnp.int32(-65536))
    lo = lax.shift_right_logical(ub, 16)
    return jnp.bitwise_or(hi, lo)


def _unpack(w):
    h = lax.bitcast_convert_type(jnp.bitwise_and(w, jnp.int32(-65536)), jnp.float32)
    hu = lax.bitcast_convert_type(lax.shift_left(w, 16), jnp.float32)
    return h, hu


def _sc_gather(table, idx):
    mesh = plsc.VectorSubcoreMesh(core_axis_name="c", subcore_axis_name="s")

    @functools.partial(
        pl.kernel,
        mesh=mesh,
        out_type=jax.ShapeDtypeStruct((_B, D), jnp.int32),
        scratch_types=[
            pltpu.VMEM((_PER_W,), jnp.int32),
            pltpu.VMEM((_CH, D), jnp.int32),
            pltpu.VMEM((_CH, D), jnp.int32),
            pltpu.SemaphoreType.DMA,
            pltpu.SemaphoreType.DMA,
            pltpu.SemaphoreType.DMA,
            pltpu.SemaphoreType.DMA,
        ],
    )
    def gather_kernel(table_hbm, idx_hbm, out_hbm, idx_all,
                      b0, b1, g0, g1, s0, s1):
        wid = lax.axis_index("s") * _NC + lax.axis_index("c")
        base = wid * _PER_W
        rows = (b0, b1)
        gsem = (g0, g1)
        ssem = (s0, s1)

        pltpu.sync_copy(idx_hbm.at[pl.ds(base, _PER_W)], idx_all)

        def fire_g(c, b):
            pltpu.async_copy(
                table_hbm.at[idx_all.at[pl.ds(c * _CH, _CH)]], rows[b], gsem[b])

        def wait_g(c, b):
            pltpu.make_async_copy(
                table_hbm.at[idx_all.at[pl.ds(c * _CH, _CH)]], rows[b],
                gsem[b]).wait()

        def fire_s(c, b):
            pltpu.async_copy(
                rows[b], out_hbm.at[pl.ds(base + c * _CH, _CH)], ssem[b])

        def wait_s(c, b):
            pltpu.make_async_copy(
                rows[b], out_hbm.at[pl.ds(base + c * _CH, _CH)], ssem[b]).wait()

        fire_g(0, 0)

        def body(i, carry):
            c = i * 2

            @pl.when(i > 0)
            def _():
                wait_s(c - 1, 1)

            fire_g(c + 1, 1)
            wait_g(c, 0)
            fire_s(c, 0)

            wait_s(c, 0)

            @pl.when(i < _NCHUNK // 2 - 1)
            def _():
                fire_g(c + 2, 0)

            wait_g(c + 1, 1)
            fire_s(c + 1, 1)
            return carry

        lax.fori_loop(0, _NCHUNK // 2, body, 0)
        wait_s(_NCHUNK - 1, 1)

    return gather_kernel(table, idx)


def _precompute(fmess, w_x, b_x, ur_t):
    grid = E // _RP

    def body(x_ref, w_ref, b_ref, u_ref, xzrh_ref, hcat_ref):
        x = x_ref[...]
        xzrh = jnp.dot(x, w_ref[...], preferred_element_type=jnp.float32) + b_ref[...]
        xzrh_ref[...] = xzrh.astype(jnp.bfloat16)
        h1 = _sigmoid(xzrh[:, :D]) * jnp.tanh(xzrh[:, 2 * D:])
        rows = lax.broadcasted_iota(jnp.int32, (_RP, 1), 0)
        keep = jnp.where((rows == 0) & (pl.program_id(0) == 0), 0.0, 1.0)
        h1 = h1 * keep
        hu1 = jnp.dot(h1, u_ref[...], preferred_element_type=jnp.float32)
        hcat_ref[...] = _pack(h1, hu1)

    return pl.pallas_call(
        body,
        grid=(grid,),
        in_specs=[
            pl.BlockSpec((_RP, D), lambda i: (i, 0)),
            pl.BlockSpec((D, 3 * D), lambda i: (0, 0)),
            pl.BlockSpec((1, 3 * D), lambda i: (0, 0)),
            pl.BlockSpec((D, D), lambda i: (0, 0)),
        ],
        out_specs=[
            pl.BlockSpec((_RP, 3 * D), lambda i: (i, 0)),
            pl.BlockSpec((_RP, D), lambda i: (i, 0)),
        ],
        out_shape=[
            jax.ShapeDtypeStruct((E, 3 * D), jnp.bfloat16),
            jax.ShapeDtypeStruct((E, D), jnp.int32),
        ],
    )(fmess, w_x, b_x, ur_t)


def _update(gathered, xzrh, wz_h_t, wh_h_t, ur_t, ur_b, last):
    grid = E // _RU

    def body(g_ref, x_ref, wz_ref, wh_ref, u_ref, ub_ref, out_ref):
        x = x_ref[...].astype(jnp.float32)
        xz = x[:, :D]
        xr = x[:, D:2 * D]
        xh = x[:, 2 * D:]
        ub = ub_ref[...]
        sum_h = jnp.zeros((_RU, D), jnp.float32)
        sum_g = jnp.zeros((_RU, D), jnp.float32)
        for k in range(K):
            hk, huk = _unpack(g_ref[k, :, :])
            sum_h = sum_h + hk
            rk = _sigmoid(xr + huk + ub)
            sum_g = sum_g + rk * hk
        z = _sigmoid(xz + jnp.dot(sum_h, wz_ref[...], preferred_element_type=jnp.float32))
        pre = jnp.tanh(xh + jnp.dot(sum_g, wh_ref[...], preferred_element_type=jnp.float32))
        h = (1.0 - z) * sum_h + z * pre
        rows = lax.broadcasted_iota(jnp.int32, (_RU, 1), 0)
        keep = jnp.where((rows == 0) & (pl.program_id(0) == 0), 0.0, 1.0)
        h = h * keep
        if last:
            out_ref[...] = h
        else:
            hu = jnp.dot(h, u_ref[...], preferred_element_type=jnp.float32)
            out_ref[...] = _pack(h, hu)

    return pl.pallas_call(
        body,
        grid=(grid,),
        in_specs=[
            pl.BlockSpec((K, _RU, D), lambda i: (0, i, 0)),
            pl.BlockSpec((_RU, 3 * D), lambda i: (i, 0)),
            pl.BlockSpec((D, D), lambda i: (0, 0)),
            pl.BlockSpec((D, D), lambda i: (0, 0)),
            pl.BlockSpec((D, D), lambda i: (0, 0)),
            pl.BlockSpec((1, D), lambda i: (0, 0)),
        ],
        out_specs=pl.BlockSpec((_RU, D), lambda i: (i, 0)),
        out_shape=jax.ShapeDtypeStruct(
            (E, D), jnp.float32 if last else jnp.int32),
    )(gathered, xzrh, wz_h_t, wh_h_t, ur_t, ur_b)


def kernel(fmess, bgraph, Wz_w, Wz_b, Wr_w, Ur_w, Ur_b, Wh_w, Wh_b):
    bg = bgraph.astype(jnp.int32)
    idx = bg.T.reshape(_B)
    w_x = jnp.concatenate([Wz_w[:, :D].T, Wr_w.T, Wh_w[:, :D].T], axis=1)
    b_x = jnp.concatenate([Wz_b, jnp.zeros((D,), jnp.float32), Wh_b]).reshape(1, 3 * D)
    ur_t = Ur_w.T
    wz_h_t = Wz_w[:, D:].T
    wh_h_t = Wh_w[:, D:].T
    ur_b = Ur_b.reshape(1, D)

    xzrh, hcat = _precompute(fmess, w_x, b_x, ur_t)
    h = None
    for d in range(2):
        gathered = _sc_gather(hcat, idx).reshape(K, E, D)
        last = d == 1
        out = _update(gathered, xzrh, wz_h_t, wh_h_t, ur_t, ur_b, last)
        if last:
            h = out
        else:
            hcat = out
    return h


# speedup vs baseline: 7.7044x; 1.0273x over previous
import functools

import jax
import jax.numpy as jnp
from jax import lax
from jax.experimental import pallas as pl
from jax.experimental.pallas import tpu as pltpu
from jax.experimental.pallas import tpu_sc as plsc

E = 160000
K = 6
D = 128

_NC = 2
_NS = 16
_NW = _NC * _NS
_CH = 120
_B = E * K
_PER_W = _B // _NW
_NCHUNK = _PER_W // _CH

_RP = 1600
_RU = 800


def _sigmoid(x):
    return 0.5 * jnp.tanh(0.5 * x) + 0.5


def _pack(h, hu):
    hb = lax.bitcast_convert_type(h, jnp.int32) + 0x8000
    ub = lax.bitcast_convert_type(hu, jnp.int32) + 0x8000
    hi = jnp.bitwise_and(hb, jnp.int32(-65536))
    lo = lax.shift_right_logical(ub, 16)
    return jnp.bitwise_or(hi, lo)


def _unpack(w):
    h = lax.bitcast_convert_type(jnp.bitwise_and(w, jnp.int32(-65536)), jnp.float32)
    hu = lax.bitcast_convert_type(lax.shift_left(w, 16), jnp.float32)
    return h, hu


def _sc_gather(table, idx):
    mesh = plsc.VectorSubcoreMesh(core_axis_name="c", subcore_axis_name="s")

    @functools.partial(
        pl.kernel,
        mesh=mesh,
        out_type=jax.ShapeDtypeStruct((_B, D), jnp.int32),
        scratch_types=[
            pltpu.VMEM((_PER_W,), jnp.int32),
            pltpu.VMEM((_CH, D), jnp.int32),
            pltpu.VMEM((_CH, D), jnp.int32),
            pltpu.SemaphoreType.DMA,
            pltpu.SemaphoreType.DMA,
            pltpu.SemaphoreType.DMA,
            pltpu.SemaphoreType.DMA,
        ],
    )
    def gather_kernel(table_hbm, idx_hbm, out_hbm, idx_all,
                      b0, b1, g0, g1, s0, s1):
        wid = lax.axis_index("s") * _NC + lax.axis_index("c")
        base = wid * _PER_W
        rows = (b0, b1)
        gsem = (g0, g1)
        ssem = (s0, s1)

        pltpu.sync_copy(idx_hbm.at[pl.ds(base, _PER_W)], idx_all)

        def fire_g(c, b):
            pltpu.async_copy(
                table_hbm.at[idx_all.at[pl.ds(c * _CH, _CH)]], rows[b], gsem[b])

        def wait_g(c, b):
            pltpu.make_async_copy(
                table_hbm.at[idx_all.at[pl.ds(c * _CH, _CH)]], rows[b],
                gsem[b]).wait()

        def fire_s(c, b):
            pltpu.async_copy(
                rows[b], out_hbm.at[pl.ds(base + c * _CH, _CH)], ssem[b])

        def wait_s(c, b):
            pltpu.make_async_copy(
                rows[b], out_hbm.at[pl.ds(base + c * _CH, _CH)], ssem[b]).wait()

        fire_g(0, 0)

        def body(i, carry):
            c = i * 2

            @pl.when(i > 0)
            def _():
                wait_s(c - 1, 1)

            fire_g(c + 1, 1)
            wait_g(c, 0)
            fire_s(c, 0)

            wait_s(c, 0)

            @pl.when(i < _NCHUNK // 2 - 1)
            def _():
                fire_g(c + 2, 0)

            wait_g(c + 1, 1)
            fire_s(c + 1, 1)
            return carry

        lax.fori_loop(0, _NCHUNK // 2, body, 0)
        wait_s(_NCHUNK - 1, 1)

    return gather_kernel(table, idx)


def _precompute(fmess, w_x, b_x, ur_t):
    grid = E // _RP

    def body(x_ref, w_ref, b_ref, u_ref, xzrh_ref, hcat_ref):
        x = x_ref[...]
        xzrh = jnp.dot(x, w_ref[...], preferred_element_type=jnp.float32) + b_ref[...]
        xzrh_ref[...] = xzrh.astype(jnp.bfloat16)
        h1 = _sigmoid(xzrh[:, :D]) * jnp.tanh(xzrh[:, 2 * D:])
        rows = lax.broadcasted_iota(jnp.int32, (_RP, 1), 0)
        keep = jnp.where((rows == 0) & (pl.program_id(0) == 0), 0.0, 1.0)
        h1 = h1 * keep
        hu1 = jnp.dot(h1, u_ref[...], preferred_element_type=jnp.float32)
        hcat_ref[...] = _pack(h1, hu1)

    return pl.pallas_call(
        body,
        grid=(grid,),
        in_specs=[
            pl.BlockSpec((_RP, D), lambda i: (i, 0)),
            pl.BlockSpec((D, 3 * D), lambda i: (0, 0)),
            pl.BlockSpec((1, 3 * D), lambda i: (0, 0)),
            pl.BlockSpec((D, D), lambda i: (0, 0)),
        ],
        out_specs=[
            pl.BlockSpec((_RP, 3 * D), lambda i: (i, 0)),
            pl.BlockSpec((_RP, D), lambda i: (i, 0)),
        ],
        out_shape=[
            jax.ShapeDtypeStruct((E, 3 * D), jnp.bfloat16),
            jax.ShapeDtypeStruct((E, D), jnp.int32),
        ],
    )(fmess, w_x, b_x, ur_t)


def _update(gathered, xzrh, wz_h_t, wh_h_t, ur_t, ur_b, last):
    grid = E // _RU

    def body(g_ref, x_ref, wz_ref, wh_ref, u_ref, ub_ref, out_ref):
        x = x_ref[...].astype(jnp.float32)
        xz = x[:, :D]
        xr = x[:, D:2 * D]
        xh = x[:, 2 * D:]
        ub = ub_ref[...]
        xr2 = 0.5 * (xr + ub)
        sum_h = jnp.zeros((_RU, D), jnp.float32)
        sum_t = jnp.zeros((_RU, D), jnp.float32)
        for k in range(K):
            hk, huk2 = _unpack(g_ref[k, :, :])
            sum_h = sum_h + hk
            tk = jnp.tanh(xr2 + huk2)
            sum_t = sum_t + tk * hk
        sum_g = 0.5 * (sum_h + sum_t)
        z = _sigmoid(xz + jnp.dot(sum_h, wz_ref[...], preferred_element_type=jnp.float32))
        pre = jnp.tanh(xh + jnp.dot(sum_g, wh_ref[...], preferred_element_type=jnp.float32))
        h = (1.0 - z) * sum_h + z * pre
        rows = lax.broadcasted_iota(jnp.int32, (_RU, 1), 0)
        keep = jnp.where((rows == 0) & (pl.program_id(0) == 0), 0.0, 1.0)
        h = h * keep
        if last:
            out_ref[...] = h
        else:
            hu = jnp.dot(h, u_ref[...], preferred_element_type=jnp.float32)
            out_ref[...] = _pack(h, hu)

    return pl.pallas_call(
        body,
        grid=(grid,),
        in_specs=[
            pl.BlockSpec((K, _RU, D), lambda i: (0, i, 0)),
            pl.BlockSpec((_RU, 3 * D), lambda i: (i, 0)),
            pl.BlockSpec((D, D), lambda i: (0, 0)),
            pl.BlockSpec((D, D), lambda i: (0, 0)),
            pl.BlockSpec((D, D), lambda i: (0, 0)),
            pl.BlockSpec((1, D), lambda i: (0, 0)),
        ],
        out_specs=pl.BlockSpec((_RU, D), lambda i: (i, 0)),
        out_shape=jax.ShapeDtypeStruct(
            (E, D), jnp.float32 if last else jnp.int32),
    )(gathered, xzrh, wz_h_t, wh_h_t, ur_t, ur_b)


def kernel(fmess, bgraph, Wz_w, Wz_b, Wr_w, Ur_w, Ur_b, Wh_w, Wh_b):
    bg = bgraph.astype(jnp.int32)
    idx = bg.T.reshape(_B)
    w_x = jnp.concatenate([Wz_w[:, :D].T, Wr_w.T, Wh_w[:, :D].T], axis=1)
    b_x = jnp.concatenate([Wz_b, jnp.zeros((D,), jnp.float32), Wh_b]).reshape(1, 3 * D)
    ur_t = 0.5 * Ur_w.T
    wz_h_t = Wz_w[:, D:].T
    wh_h_t = Wh_w[:, D:].T
    ur_b = Ur_b.reshape(1, D)

    xzrh, hcat = _precompute(fmess, w_x, b_x, ur_t)
    h = None
    for d in range(2):
        gathered = _sc_gather(hcat, idx).reshape(K, E, D)
        last = d == 1
        out = _update(gathered, xzrh, wz_h_t, wh_h_t, ur_t, ur_b, last)
        if last:
            h = out
        else:
            hcat = out
    return h


# speedup vs baseline: 7.7865x; 1.0107x over previous
import functools

import jax
import jax.numpy as jnp
from jax import lax
from jax.experimental import pallas as pl
from jax.experimental.pallas import tpu as pltpu
from jax.experimental.pallas import tpu_sc as plsc

E = 160000
K = 6
D = 128

_NC = 2
_NS = 16
_NW = _NC * _NS
_CH = 120
_B = E * K
_PER_W = _B // _NW
_NCHUNK = _PER_W // _CH

_RP = 1600
_RU = 800


def _sigmoid(x):
    return 0.5 * jnp.tanh(0.5 * x) + 0.5


def _pack(h, hu):
    hb = lax.bitcast_convert_type(h, jnp.int32) + 0x8000
    ub = lax.bitcast_convert_type(hu, jnp.int32) + 0x8000
    hi = jnp.bitwise_and(hb, jnp.int32(-65536))
    lo = lax.shift_right_logical(ub, 16)
    return jnp.bitwise_or(hi, lo)


def _unpack(w):
    h = lax.bitcast_convert_type(jnp.bitwise_and(w, jnp.int32(-65536)), jnp.float32)
    hu = lax.bitcast_convert_type(lax.shift_left(w, 16), jnp.float32)
    return h, hu


def _sc_gather(table, idx):
    mesh = plsc.VectorSubcoreMesh(core_axis_name="c", subcore_axis_name="s")

    @functools.partial(
        pl.kernel,
        mesh=mesh,
        out_type=jax.ShapeDtypeStruct((_B, D), jnp.int32),
        scratch_types=[
            pltpu.VMEM((_PER_W,), jnp.int32),
            pltpu.VMEM((_CH, D), jnp.int32),
            pltpu.VMEM((_CH, D), jnp.int32),
            pltpu.VMEM((_CH, D), jnp.int32),
            pltpu.VMEM((_CH, D), jnp.int32),
            pltpu.SemaphoreType.DMA,
            pltpu.SemaphoreType.DMA,
            pltpu.SemaphoreType.DMA,
            pltpu.SemaphoreType.DMA,
            pltpu.SemaphoreType.DMA,
            pltpu.SemaphoreType.DMA,
            pltpu.SemaphoreType.DMA,
            pltpu.SemaphoreType.DMA,
        ],
    )
    def gather_kernel(table_hbm, idx_hbm, out_hbm, idx_all,
                      b0, b1, b2, b3, g0, g1, g2, g3, s0, s1, s2, s3):
        wid = lax.axis_index("s") * _NC + lax.axis_index("c")
        base = wid * _PER_W
        rows = (b0, b1, b2, b3)
        gsem = (g0, g1, g2, g3)
        ssem = (s0, s1, s2, s3)

        pltpu.sync_copy(idx_hbm.at[pl.ds(base, _PER_W)], idx_all)

        def fire_g(c, b):
            pltpu.async_copy(
                table_hbm.at[idx_all.at[pl.ds(c * _CH, _CH)]], rows[b], gsem[b])

        def wait_g(c, b):
            pltpu.make_async_copy(
                table_hbm.at[idx_all.at[pl.ds(c * _CH, _CH)]], rows[b],
                gsem[b]).wait()

        def fire_s(c, b):
            pltpu.async_copy(
                rows[b], out_hbm.at[pl.ds(base + c * _CH, _CH)], ssem[b])

        def wait_s(c, b):
            pltpu.make_async_copy(
                rows[b], out_hbm.at[pl.ds(base + c * _CH, _CH)], ssem[b]).wait()

        fire_g(0, 0)
        fire_g(1, 1)

        def body(i, carry):
            c = i * 4

            @pl.when(i > 0)
            def _():
                wait_s(c - 2, 2)

            fire_g(c + 2, 2)
            wait_g(c, 0)
            fire_s(c, 0)

            @pl.when(i > 0)
            def _():
                wait_s(c - 1, 3)

            fire_g(c + 3, 3)
            wait_g(c + 1, 1)
            fire_s(c + 1, 1)

            wait_s(c, 0)
            fire_g(c + 4, 0)
            wait_g(c + 2, 2)
            fire_s(c + 2, 2)

            wait_s(c + 1, 1)
            fire_g(c + 5, 1)
            wait_g(c + 3, 3)
            fire_s(c + 3, 3)
            return carry

        nbody = (_NCHUNK - 2) // 4
        lax.fori_loop(0, nbody, body, 0)
        clast = nbody * 4
        wait_g(clast, 0)
        fire_s(clast, 0)
        wait_g(clast + 1, 1)
        fire_s(clast + 1, 1)
        wait_s(clast - 2, 2)
        wait_s(clast - 1, 3)
        wait_s(clast, 0)
        wait_s(clast + 1, 1)

    return gather_kernel(table, idx)


def _precompute(fmess, w_x, b_x, ur_t):
    grid = E // _RP

    def body(x_ref, w_ref, b_ref, u_ref, xzrh_ref, hcat_ref):
        x = x_ref[...]
        xzrh = jnp.dot(x, w_ref[...], preferred_element_type=jnp.float32) + b_ref[...]
        xzrh_ref[...] = xzrh.astype(jnp.bfloat16)
        h1 = _sigmoid(xzrh[:, :D]) * jnp.tanh(xzrh[:, 2 * D:])
        rows = lax.broadcasted_iota(jnp.int32, (_RP, 1), 0)
        keep = jnp.where((rows == 0) & (pl.program_id(0) == 0), 0.0, 1.0)
        h1 = h1 * keep
        hu1 = jnp.dot(h1, u_ref[...], preferred_element_type=jnp.float32)
        hcat_ref[...] = _pack(h1, hu1)

    return pl.pallas_call(
        body,
        grid=(grid,),
        in_specs=[
            pl.BlockSpec((_RP, D), lambda i: (i, 0)),
            pl.BlockSpec((D, 3 * D), lambda i: (0, 0)),
            pl.BlockSpec((1, 3 * D), lambda i: (0, 0)),
            pl.BlockSpec((D, D), lambda i: (0, 0)),
        ],
        out_specs=[
            pl.BlockSpec((_RP, 3 * D), lambda i: (i, 0)),
            pl.BlockSpec((_RP, D), lambda i: (i, 0)),
        ],
        out_shape=[
            jax.ShapeDtypeStruct((E, 3 * D), jnp.bfloat16),
            jax.ShapeDtypeStruct((E, D), jnp.int32),
        ],
    )(fmess, w_x, b_x, ur_t)


def _update(gathered, xzrh, wz_h_t, wh_h_t, ur_t, ur_b, last):
    grid = E // _RU

    def body(g_ref, x_ref, wz_ref, wh_ref, u_ref, ub_ref, out_ref):
        x = x_ref[...].astype(jnp.float32)
        xz = x[:, :D]
        xr = x[:, D:2 * D]
        xh = x[:, 2 * D:]
        ub = ub_ref[...]
        xr2 = 0.5 * (xr + ub)
        sum_h = jnp.zeros((_RU, D), jnp.float32)
        sum_t = jnp.zeros((_RU, D), jnp.float32)
        for k in range(K):
            hk, huk2 = _unpack(g_ref[k, :, :])
            sum_h = sum_h + hk
            tk = jnp.tanh(xr2 + huk2)
            sum_t = sum_t + tk * hk
        sum_g = 0.5 * (sum_h + sum_t)
        z = _sigmoid(xz + jnp.dot(sum_h, wz_ref[...], preferred_element_type=jnp.float32))
        pre = jnp.tanh(xh + jnp.dot(sum_g, wh_ref[...], preferred_element_type=jnp.float32))
        h = (1.0 - z) * sum_h + z * pre
        rows = lax.broadcasted_iota(jnp.int32, (_RU, 1), 0)
        keep = jnp.where((rows == 0) & (pl.program_id(0) == 0), 0.0, 1.0)
        h = h * keep
        if last:
            out_ref[...] = h
        else:
            hu = jnp.dot(h, u_ref[...], preferred_element_type=jnp.float32)
            out_ref[...] = _pack(h, hu)

    return pl.pallas_call(
        body,
        grid=(grid,),
        in_specs=[
            pl.BlockSpec((K, _RU, D), lambda i: (0, i, 0)),
            pl.BlockSpec((_RU, 3 * D), lambda i: (i, 0)),
            pl.BlockSpec((D, D), lambda i: (0, 0)),
            pl.BlockSpec((D, D), lambda i: (0, 0)),
            pl.BlockSpec((D, D), lambda i: (0, 0)),
            pl.BlockSpec((1, D), lambda i: (0, 0)),
        ],
        out_specs=pl.BlockSpec((_RU, D), lambda i: (i, 0)),
        out_shape=jax.ShapeDtypeStruct(
            (E, D), jnp.float32 if last else jnp.int32),
    )(gathered, xzrh, wz_h_t, wh_h_t, ur_t, ur_b)


def kernel(fmess, bgraph, Wz_w, Wz_b, Wr_w, Ur_w, Ur_b, Wh_w, Wh_b):
    bg = bgraph.astype(jnp.int32)
    idx = bg.T.reshape(_B)
    w_x = jnp.concatenate([Wz_w[:, :D].T, Wr_w.T, Wh_w[:, :D].T], axis=1)
    b_x = jnp.concatenate([Wz_b, jnp.zeros((D,), jnp.float32), Wh_b]).reshape(1, 3 * D)
    ur_t = 0.5 * Ur_w.T
    wz_h_t = Wz_w[:, D:].T
    wh_h_t = Wh_w[:, D:].T
    ur_b = Ur_b.reshape(1, D)

    xzrh, hcat = _precompute(fmess, w_x, b_x, ur_t)
    h = None
    for d in range(2):
        gathered = _sc_gather(hcat, idx).reshape(K, E, D)
        last = d == 1
        out = _update(gathered, xzrh, wz_h_t, wh_h_t, ur_t, ur_b, last)
        if last:
            h = out
        else:
            hcat = out
    return h


# speedup vs baseline: 8.5495x; 1.0980x over previous
import functools

import jax
import jax.numpy as jnp
from jax import lax
from jax.experimental import pallas as pl
from jax.experimental.pallas import tpu as pltpu
from jax.experimental.pallas import tpu_sc as plsc

E = 160000
K = 6
D = 128

_NC = 2
_NS = 16
_NW = _NC * _NS
_CH = 120
_B = E * K
_PER_W = _B // _NW
_NCHUNK = _PER_W // _CH

_RP = 1600
_RU = 1600


def _sigmoid(x):
    return 0.5 * jnp.tanh(0.5 * x) + 0.5


def _pack(h, hu):
    hb = lax.bitcast_convert_type(h, jnp.int32) + 0x8000
    ub = lax.bitcast_convert_type(hu, jnp.int32) + 0x8000
    hi = jnp.bitwise_and(ub, jnp.int32(-65536))
    lo = lax.shift_right_logical(hb, 16)
    return jnp.bitwise_or(hi, lo)


def _unpack(w):
    h = lax.bitcast_convert_type(lax.shift_left(w, 16), jnp.float32)
    hu = lax.bitcast_convert_type(w, jnp.float32)
    return h, hu


def _sc_gather(table, idx):
    mesh = plsc.VectorSubcoreMesh(core_axis_name="c", subcore_axis_name="s")

    @functools.partial(
        pl.kernel,
        mesh=mesh,
        out_type=jax.ShapeDtypeStruct((_B, D), jnp.int32),
        scratch_types=[
            pltpu.VMEM((_PER_W,), jnp.int32),
            pltpu.VMEM((_CH, D), jnp.int32),
            pltpu.VMEM((_CH, D), jnp.int32),
            pltpu.VMEM((_CH, D), jnp.int32),
            pltpu.VMEM((_CH, D), jnp.int32),
            pltpu.SemaphoreType.DMA,
            pltpu.SemaphoreType.DMA,
            pltpu.SemaphoreType.DMA,
            pltpu.SemaphoreType.DMA,
            pltpu.SemaphoreType.DMA,
            pltpu.SemaphoreType.DMA,
            pltpu.SemaphoreType.DMA,
            pltpu.SemaphoreType.DMA,
        ],
    )
    def gather_kernel(table_hbm, idx_hbm, out_hbm, idx_all,
                      b0, b1, b2, b3, g0, g1, g2, g3, s0, s1, s2, s3):
        wid = lax.axis_index("s") * _NC + lax.axis_index("c")
        base = wid * _PER_W
        rows = (b0, b1, b2, b3)
        gsem = (g0, g1, g2, g3)
        ssem = (s0, s1, s2, s3)

        pltpu.sync_copy(idx_hbm.at[pl.ds(base, _PER_W)], idx_all)

        def fire_g(c, b):
            pltpu.async_copy(
                table_hbm.at[idx_all.at[pl.ds(c * _CH, _CH)]], rows[b], gsem[b])

        def wait_g(c, b):
            pltpu.make_async_copy(
                table_hbm.at[idx_all.at[pl.ds(c * _CH, _CH)]], rows[b],
                gsem[b]).wait()

        def fire_s(c, b):
            pltpu.async_copy(
                rows[b], out_hbm.at[pl.ds(base + c * _CH, _CH)], ssem[b])

        def wait_s(c, b):
            pltpu.make_async_copy(
                rows[b], out_hbm.at[pl.ds(base + c * _CH, _CH)], ssem[b]).wait()

        fire_g(0, 0)
        fire_g(1, 1)

        def body(i, carry):
            c = i * 4

            @pl.when(i > 0)
            def _():
                wait_s(c - 2, 2)

            fire_g(c + 2, 2)
            wait_g(c, 0)
            fire_s(c, 0)

            @pl.when(i > 0)
            def _():
                wait_s(c - 1, 3)

            fire_g(c + 3, 3)
            wait_g(c + 1, 1)
            fire_s(c + 1, 1)

            wait_s(c, 0)
            fire_g(c + 4, 0)
            wait_g(c + 2, 2)
            fire_s(c + 2, 2)

            wait_s(c + 1, 1)
            fire_g(c + 5, 1)
            wait_g(c + 3, 3)
            fire_s(c + 3, 3)
            return carry

        nbody = (_NCHUNK - 2) // 4
        lax.fori_loop(0, nbody, body, 0)
        clast = nbody * 4
        wait_g(clast, 0)
        fire_s(clast, 0)
        wait_g(clast + 1, 1)
        fire_s(clast + 1, 1)
        wait_s(clast - 2, 2)
        wait_s(clast - 1, 3)
        wait_s(clast, 0)
        wait_s(clast + 1, 1)

    return gather_kernel(table, idx)


def _precompute(fmess, w_x, b_x, ur_t):
    grid = E // _RP

    def body(x_ref, w_ref, b_ref, u_ref, xzrh_ref, hcat_ref):
        x = x_ref[...]
        xzrh = jnp.dot(x, w_ref[...], preferred_element_type=jnp.float32) + b_ref[...]
        xzrh_ref[...] = xzrh.astype(jnp.bfloat16)
        h1 = _sigmoid(xzrh[:, :D]) * jnp.tanh(xzrh[:, 2 * D:])
        rows = lax.broadcasted_iota(jnp.int32, (_RP, 1), 0)
        keep = jnp.where((rows == 0) & (pl.program_id(0) == 0), 0.0, 1.0)
        h1 = h1 * keep
        hu1 = jnp.dot(h1, u_ref[...], preferred_element_type=jnp.float32)
        hcat_ref[...] = _pack(h1, hu1)

    return pl.pallas_call(
        body,
        grid=(grid,),
        in_specs=[
            pl.BlockSpec((_RP, D), lambda i: (i, 0)),
            pl.BlockSpec((D, 3 * D), lambda i: (0, 0)),
            pl.BlockSpec((1, 3 * D), lambda i: (0, 0)),
            pl.BlockSpec((D, D), lambda i: (0, 0)),
        ],
        out_specs=[
            pl.BlockSpec((_RP, 3 * D), lambda i: (i, 0)),
            pl.BlockSpec((_RP, D), lambda i: (i, 0)),
        ],
        out_shape=[
            jax.ShapeDtypeStruct((E, 3 * D), jnp.bfloat16),
            jax.ShapeDtypeStruct((E, D), jnp.int32),
        ],
    )(fmess, w_x, b_x, ur_t)


def _update(gathered, xzrh, wz_h_t, wh_h_t, ur_t, ur_b, last):
    grid = E // _RU

    def body(g_ref, x_ref, wz_ref, wh_ref, u_ref, ub_ref, out_ref):
        x = x_ref[...].astype(jnp.float32)
        xz = x[:, :D]
        xr = x[:, D:2 * D]
        xh = x[:, 2 * D:]
        ub = ub_ref[...]
        xr2 = 0.5 * (xr + ub)
        sum_h = jnp.zeros((_RU, D), jnp.float32)
        sum_t = jnp.zeros((_RU, D), jnp.float32)
        for k in range(K):
            hk, huk2 = _unpack(g_ref[k, :, :])
            sum_h = sum_h + hk
            tk = jnp.tanh(xr2 + huk2)
            sum_t = sum_t + tk * hk
        sum_g = 0.5 * (sum_h + sum_t)
        z = _sigmoid(xz + jnp.dot(sum_h, wz_ref[...], preferred_element_type=jnp.float32))
        pre = jnp.tanh(xh + jnp.dot(sum_g, wh_ref[...], preferred_element_type=jnp.float32))
        h = (1.0 - z) * sum_h + z * pre
        rows = lax.broadcasted_iota(jnp.int32, (_RU, 1), 0)
        keep = jnp.where((rows == 0) & (pl.program_id(0) == 0), 0.0, 1.0)
        h = h * keep
        if last:
            out_ref[...] = h
        else:
            hu = jnp.dot(h, u_ref[...], preferred_element_type=jnp.float32)
            out_ref[...] = _pack(h, hu)

    return pl.pallas_call(
        body,
        grid=(grid,),
        in_specs=[
            pl.BlockSpec((K, _RU, D), lambda i: (0, i, 0)),
            pl.BlockSpec((_RU, 3 * D), lambda i: (i, 0)),
            pl.BlockSpec((D, D), lambda i: (0, 0)),
            pl.BlockSpec((D, D), lambda i: (0, 0)),
            pl.BlockSpec((D, D), lambda i: (0, 0)),
            pl.BlockSpec((1, D), lambda i: (0, 0)),
        ],
        out_specs=pl.BlockSpec((_RU, D), lambda i: (i, 0)),
        out_shape=jax.ShapeDtypeStruct(
            (E, D), jnp.float32 if last else jnp.int32),
    )(gathered, xzrh, wz_h_t, wh_h_t, ur_t, ur_b)


def kernel(fmess, bgraph, Wz_w, Wz_b, Wr_w, Ur_w, Ur_b, Wh_w, Wh_b):
    bg = bgraph.astype(jnp.int32)
    idx = bg.T.reshape(_B)
    w_x = jnp.concatenate([Wz_w[:, :D].T, Wr_w.T, Wh_w[:, :D].T], axis=1)
    b_x = jnp.concatenate([Wz_b, jnp.zeros((D,), jnp.float32), Wh_b]).reshape(1, 3 * D)
    ur_t = 0.5 * Ur_w.T
    wz_h_t = Wz_w[:, D:].T
    wh_h_t = Wh_w[:, D:].T
    ur_b = Ur_b.reshape(1, D)

    xzrh, hcat = _precompute(fmess, w_x, b_x, ur_t)
    h = None
    for d in range(2):
        gathered = _sc_gather(hcat, idx).reshape(K, E, D)
        last = d == 1
        out = _update(gathered, xzrh, wz_h_t, wh_h_t, ur_t, ur_b, last)
        if last:
            h = out
        else:
            hcat = out
    return h


# speedup vs baseline: 8.7653x; 1.0252x over previous
import functools

import jax
import jax.numpy as jnp
from jax import lax
from jax.experimental import pallas as pl
from jax.experimental.pallas import tpu as pltpu
from jax.experimental.pallas import tpu_sc as plsc

E = 160000
K = 6
D = 128

_NC = 2
_NS = 16
_NW = _NC * _NS
_CH = 120
_B = E * K
_PER_W = _B // _NW
_NCHUNK = _PER_W // _CH

_RP = 1600
_RU = 3200


def _sigmoid(x):
    return 0.5 * jnp.tanh(0.5 * x) + 0.5


def _pack(h, hu):
    hb = lax.bitcast_convert_type(h, jnp.int32) + 0x8000
    ub = lax.bitcast_convert_type(hu, jnp.int32) + 0x8000
    hi = jnp.bitwise_and(ub, jnp.int32(-65536))
    lo = lax.shift_right_logical(hb, 16)
    return jnp.bitwise_or(hi, lo)


def _unpack(w):
    h = lax.bitcast_convert_type(lax.shift_left(w, 16), jnp.float32)
    hu = lax.bitcast_convert_type(w, jnp.float32)
    return h, hu


def _sc_gather(table, idx):
    mesh = plsc.VectorSubcoreMesh(core_axis_name="c", subcore_axis_name="s")

    @functools.partial(
        pl.kernel,
        mesh=mesh,
        out_type=jax.ShapeDtypeStruct((_B, D), jnp.int32),
        scratch_types=[
            pltpu.VMEM((_PER_W,), jnp.int32),
            pltpu.VMEM((_CH, D), jnp.int32),
            pltpu.VMEM((_CH, D), jnp.int32),
            pltpu.VMEM((_CH, D), jnp.int32),
            pltpu.VMEM((_CH, D), jnp.int32),
            pltpu.SemaphoreType.DMA,
            pltpu.SemaphoreType.DMA,
            pltpu.SemaphoreType.DMA,
            pltpu.SemaphoreType.DMA,
            pltpu.SemaphoreType.DMA,
            pltpu.SemaphoreType.DMA,
            pltpu.SemaphoreType.DMA,
            pltpu.SemaphoreType.DMA,
        ],
    )
    def gather_kernel(table_hbm, idx_hbm, out_hbm, idx_all,
                      b0, b1, b2, b3, g0, g1, g2, g3, s0, s1, s2, s3):
        wid = lax.axis_index("s") * _NC + lax.axis_index("c")
        base = wid * _PER_W
        rows = (b0, b1, b2, b3)
        gsem = (g0, g1, g2, g3)
        ssem = (s0, s1, s2, s3)

        pltpu.sync_copy(idx_hbm.at[pl.ds(base, _PER_W)], idx_all)

        def fire_g(c, b):
            pltpu.async_copy(
                table_hbm.at[idx_all.at[pl.ds(c * _CH, _CH)]], rows[b], gsem[b])

        def wait_g(c, b):
            pltpu.make_async_copy(
                table_hbm.at[idx_all.at[pl.ds(c * _CH, _CH)]], rows[b],
                gsem[b]).wait()

        def fire_s(c, b):
            pltpu.async_copy(
                rows[b], out_hbm.at[pl.ds(base + c * _CH, _CH)], ssem[b])

        def wait_s(c, b):
            pltpu.make_async_copy(
                rows[b], out_hbm.at[pl.ds(base + c * _CH, _CH)], ssem[b]).wait()

        fire_g(0, 0)
        fire_g(1, 1)

        def body(i, carry):
            c = i * 4

            @pl.when(i > 0)
            def _():
                wait_s(c - 2, 2)

            fire_g(c + 2, 2)
            wait_g(c, 0)
            fire_s(c, 0)

            @pl.when(i > 0)
            def _():
                wait_s(c - 1, 3)

            fire_g(c + 3, 3)
            wait_g(c + 1, 1)
            fire_s(c + 1, 1)

            wait_s(c, 0)
            fire_g(c + 4, 0)
            wait_g(c + 2, 2)
            fire_s(c + 2, 2)

            wait_s(c + 1, 1)
            fire_g(c + 5, 1)
            wait_g(c + 3, 3)
            fire_s(c + 3, 3)
            return carry

        nbody = (_NCHUNK - 2) // 4
        lax.fori_loop(0, nbody, body, 0)
        clast = nbody * 4
        wait_g(clast, 0)
        fire_s(clast, 0)
        wait_g(clast + 1, 1)
        fire_s(clast + 1, 1)
        wait_s(clast - 2, 2)
        wait_s(clast - 1, 3)
        wait_s(clast, 0)
        wait_s(clast + 1, 1)

    return gather_kernel(table, idx)


def _precompute(fmess, w_x, b_x, ur_t):
    grid = E // _RP

    def body(x_ref, w_ref, b_ref, u_ref, xzrh_ref, hcat_ref):
        x = x_ref[...]
        xzrh = jnp.dot(x, w_ref[...], preferred_element_type=jnp.float32) + b_ref[...]
        xzrh_ref[...] = xzrh.astype(jnp.bfloat16)
        h1 = _sigmoid(xzrh[:, :D]) * jnp.tanh(xzrh[:, 2 * D:])
        rows = lax.broadcasted_iota(jnp.int32, (_RP, 1), 0)
        keep = jnp.where((rows == 0) & (pl.program_id(0) == 0), 0.0, 1.0)
        h1 = h1 * keep
        hu1 = jnp.dot(h1, u_ref[...], preferred_element_type=jnp.float32)
        hcat_ref[...] = _pack(h1, hu1)

    return pl.pallas_call(
        body,
        grid=(grid,),
        in_specs=[
            pl.BlockSpec((_RP, D), lambda i: (i, 0)),
            pl.BlockSpec((D, 3 * D), lambda i: (0, 0)),
            pl.BlockSpec((1, 3 * D), lambda i: (0, 0)),
            pl.BlockSpec((D, D), lambda i: (0, 0)),
        ],
        out_specs=[
            pl.BlockSpec((_RP, 3 * D), lambda i: (i, 0)),
            pl.BlockSpec((_RP, D), lambda i: (i, 0)),
        ],
        out_shape=[
            jax.ShapeDtypeStruct((E, 3 * D), jnp.bfloat16),
            jax.ShapeDtypeStruct((E, D), jnp.int32),
        ],
    )(fmess, w_x, b_x, ur_t)


def _update(gathered, xzrh, wz_h_t, wh_h_t, ur_t, ur_b, last):
    grid = E // _RU

    def body(g_ref, x_ref, wz_ref, wh_ref, u_ref, ub_ref, out_ref):
        x = x_ref[...].astype(jnp.float32)
        xz = x[:, :D]
        xr = x[:, D:2 * D]
        xh = x[:, 2 * D:]
        ub = ub_ref[...]
        xr2 = 0.5 * (xr + ub)
        sum_h = jnp.zeros((_RU, D), jnp.float32)
        sum_t = jnp.zeros((_RU, D), jnp.float32)
        for k in range(K):
            hk, huk2 = _unpack(g_ref[k, :, :])
            sum_h = sum_h + hk
            tk = jnp.tanh(xr2 + huk2)
            sum_t = sum_t + tk * hk
        sum_g = 0.5 * (sum_h + sum_t)
        z = _sigmoid(xz + jnp.dot(sum_h, wz_ref[...], preferred_element_type=jnp.float32))
        pre = jnp.tanh(xh + jnp.dot(sum_g, wh_ref[...], preferred_element_type=jnp.float32))
        h = (1.0 - z) * sum_h + z * pre
        rows = lax.broadcasted_iota(jnp.int32, (_RU, 1), 0)
        keep = jnp.where((rows == 0) & (pl.program_id(0) == 0), 0.0, 1.0)
        h = h * keep
        if last:
            out_ref[...] = h
        else:
            hu = jnp.dot(h, u_ref[...], preferred_element_type=jnp.float32)
            out_ref[...] = _pack(h, hu)

    return pl.pallas_call(
        body,
        grid=(grid,),
        in_specs=[
            pl.BlockSpec((K, _RU, D), lambda i: (0, i, 0)),
            pl.BlockSpec((_RU, 3 * D), lambda i: (i, 0)),
            pl.BlockSpec((D, D), lambda i: (0, 0)),
            pl.BlockSpec((D, D), lambda i: (0, 0)),
            pl.BlockSpec((D, D), lambda i: (0, 0)),
            pl.BlockSpec((1, D), lambda i: (0, 0)),
        ],
        out_specs=pl.BlockSpec((_RU, D), lambda i: (i, 0)),
        out_shape=jax.ShapeDtypeStruct(
            (E, D), jnp.float32 if last else jnp.int32),
    )(gathered, xzrh, wz_h_t, wh_h_t, ur_t, ur_b)


def kernel(fmess, bgraph, Wz_w, Wz_b, Wr_w, Ur_w, Ur_b, Wh_w, Wh_b):
    bg = bgraph.astype(jnp.int32)
    idx = bg.T.reshape(_B)
    w_x = jnp.concatenate([Wz_w[:, :D].T, Wr_w.T, Wh_w[:, :D].T], axis=1)
    b_x = jnp.concatenate([Wz_b, jnp.zeros((D,), jnp.float32), Wh_b]).reshape(1, 3 * D)
    ur_t = 0.5 * Ur_w.T
    wz_h_t = Wz_w[:, D:].T
    wh_h_t = Wh_w[:, D:].T
    ur_b = Ur_b.reshape(1, D)

    xzrh, hcat = _precompute(fmess, w_x, b_x, ur_t)
    h = None
    for d in range(2):
        gathered = _sc_gather(hcat, idx).reshape(K, E, D)
        last = d == 1
        out = _update(gathered, xzrh, wz_h_t, wh_h_t, ur_t, ur_b, last)
        if last:
            h = out
        else:
            hcat = out
    return h


# speedup vs baseline: 8.8581x; 1.0106x over previous
import functools

import jax
import jax.numpy as jnp
from jax import lax
from jax.experimental import pallas as pl
from jax.experimental.pallas import tpu as pltpu
from jax.experimental.pallas import tpu_sc as plsc

E = 160000
K = 6
D = 128

_NC = 2
_NS = 16
_NW = _NC * _NS
_CH = 120
_B = E * K
_PER_W = _B // _NW
_NCHUNK = _PER_W // _CH

_RP = 1600
_RU = 3200


def _sigmoid(x):
    return 0.5 * jnp.tanh(0.5 * x) + 0.5


def _pack(h, hu):
    hb = lax.bitcast_convert_type(h, jnp.int32) + 0x8000
    ub = lax.bitcast_convert_type(hu, jnp.int32) + 0x8000
    hi = jnp.bitwise_and(ub, jnp.int32(-65536))
    lo = lax.shift_right_logical(hb, 16)
    return jnp.bitwise_or(hi, lo)


def _unpack(w):
    h = lax.bitcast_convert_type(lax.shift_left(w, 16), jnp.float32)
    hu = lax.bitcast_convert_type(w, jnp.float32)
    return h, hu


def _sc_gather(table, idx):
    mesh = plsc.VectorSubcoreMesh(core_axis_name="c", subcore_axis_name="s")

    @functools.partial(
        pl.kernel,
        mesh=mesh,
        out_type=jax.ShapeDtypeStruct((_B, D), jnp.int32),
        scratch_types=[
            pltpu.VMEM((_PER_W,), jnp.int32),
            pltpu.VMEM((_CH, D), jnp.int32),
            pltpu.VMEM((_CH, D), jnp.int32),
            pltpu.VMEM((_CH, D), jnp.int32),
            pltpu.VMEM((_CH, D), jnp.int32),
            pltpu.SemaphoreType.DMA,
            pltpu.SemaphoreType.DMA,
            pltpu.SemaphoreType.DMA,
            pltpu.SemaphoreType.DMA,
            pltpu.SemaphoreType.DMA,
            pltpu.SemaphoreType.DMA,
            pltpu.SemaphoreType.DMA,
            pltpu.SemaphoreType.DMA,
        ],
    )
    def gather_kernel(table_hbm, idx_hbm, out_hbm, idx_all,
                      b0, b1, b2, b3, g0, g1, g2, g3, s0, s1, s2, s3):
        wid = lax.axis_index("s") * _NC + lax.axis_index("c")
        base = wid * _PER_W
        rows = (b0, b1, b2, b3)
        gsem = (g0, g1, g2, g3)
        ssem = (s0, s1, s2, s3)

        pltpu.sync_copy(idx_hbm.at[pl.ds(base, _PER_W)], idx_all)

        def fire_g(c, b):
            pltpu.async_copy(
                table_hbm.at[idx_all.at[pl.ds(c * _CH, _CH)]], rows[b], gsem[b])

        def wait_g(c, b):
            pltpu.make_async_copy(
                table_hbm.at[idx_all.at[pl.ds(c * _CH, _CH)]], rows[b],
                gsem[b]).wait()

        def fire_s(c, b):
            pltpu.async_copy(
                rows[b], out_hbm.at[pl.ds(base + c * _CH, _CH)], ssem[b])

        def wait_s(c, b):
            pltpu.make_async_copy(
                rows[b], out_hbm.at[pl.ds(base + c * _CH, _CH)], ssem[b]).wait()

        fire_g(0, 0)
        fire_g(1, 1)

        def body(i, carry):
            c = i * 4

            @pl.when(i > 0)
            def _():
                wait_s(c - 2, 2)

            fire_g(c + 2, 2)
            wait_g(c, 0)
            fire_s(c, 0)

            @pl.when(i > 0)
            def _():
                wait_s(c - 1, 3)

            fire_g(c + 3, 3)
            wait_g(c + 1, 1)
            fire_s(c + 1, 1)

            wait_s(c, 0)
            fire_g(c + 4, 0)
            wait_g(c + 2, 2)
            fire_s(c + 2, 2)

            wait_s(c + 1, 1)
            fire_g(c + 5, 1)
            wait_g(c + 3, 3)
            fire_s(c + 3, 3)
            return carry

        nbody = (_NCHUNK - 2) // 4
        lax.fori_loop(0, nbody, body, 0)
        clast = nbody * 4
        wait_g(clast, 0)
        fire_s(clast, 0)
        wait_g(clast + 1, 1)
        fire_s(clast + 1, 1)
        wait_s(clast - 2, 2)
        wait_s(clast - 1, 3)
        wait_s(clast, 0)
        wait_s(clast + 1, 1)

    return gather_kernel(table, idx)


def _precompute(fmess, w_x, b_x, ur_t):
    grid = E // _RP

    def body(x_ref, w_ref, b_ref, u_ref, xzrh_ref, hcat_ref):
        x = x_ref[...]
        xzrh = jnp.dot(x, w_ref[...], preferred_element_type=jnp.float32) + b_ref[...]
        xzrh_ref[...] = xzrh.astype(jnp.bfloat16)
        h1 = _sigmoid(xzrh[:, :D]) * jnp.tanh(xzrh[:, 2 * D:])
        rows = lax.broadcasted_iota(jnp.int32, (_RP, 1), 0)
        keep = jnp.where((rows == 0) & (pl.program_id(0) == 0), 0.0, 1.0)
        h1 = h1 * keep
        hu1 = jnp.dot(h1.astype(jnp.bfloat16), u_ref[...].astype(jnp.bfloat16),
                      preferred_element_type=jnp.float32)
        hcat_ref[...] = _pack(h1, hu1)

    return pl.pallas_call(
        body,
        grid=(grid,),
        in_specs=[
            pl.BlockSpec((_RP, D), lambda i: (i, 0)),
            pl.BlockSpec((D, 3 * D), lambda i: (0, 0)),
            pl.BlockSpec((1, 3 * D), lambda i: (0, 0)),
            pl.BlockSpec((D, D), lambda i: (0, 0)),
        ],
        out_specs=[
            pl.BlockSpec((_RP, 3 * D), lambda i: (i, 0)),
            pl.BlockSpec((_RP, D), lambda i: (i, 0)),
        ],
        out_shape=[
            jax.ShapeDtypeStruct((E, 3 * D), jnp.bfloat16),
            jax.ShapeDtypeStruct((E, D), jnp.int32),
        ],
    )(fmess, w_x, b_x, ur_t)


def _update(gathered, xzrh, wz_h_t, wh_h_t, ur_t, ur_b, last):
    grid = E // _RU

    def body(g_ref, x_ref, wz_ref, wh_ref, u_ref, ub_ref, out_ref):
        x = x_ref[...].astype(jnp.float32)
        xz = x[:, :D]
        xr = x[:, D:2 * D]
        xh = x[:, 2 * D:]
        ub = ub_ref[...]
        xr2 = 0.5 * (xr + ub)
        sum_h = jnp.zeros((_RU, D), jnp.float32)
        sum_t = jnp.zeros((_RU, D), jnp.float32)
        for k in range(K):
            hk, huk2 = _unpack(g_ref[k, :, :])
            sum_h = sum_h + hk
            tk = jnp.tanh(xr2 + huk2)
            sum_t = sum_t + tk * hk
        sum_g = 0.5 * (sum_h + sum_t)
        z = _sigmoid(xz + jnp.dot(sum_h.astype(jnp.bfloat16), wz_ref[...].astype(jnp.bfloat16), preferred_element_type=jnp.float32))
        pre = jnp.tanh(xh + jnp.dot(sum_g.astype(jnp.bfloat16), wh_ref[...].astype(jnp.bfloat16), preferred_element_type=jnp.float32))
        h = (1.0 - z) * sum_h + z * pre
        rows = lax.broadcasted_iota(jnp.int32, (_RU, 1), 0)
        keep = jnp.where((rows == 0) & (pl.program_id(0) == 0), 0.0, 1.0)
        h = h * keep
        if last:
            out_ref[...] = h
        else:
            hu = jnp.dot(h.astype(jnp.bfloat16), u_ref[...].astype(jnp.bfloat16),
                         preferred_element_type=jnp.float32)
            out_ref[...] = _pack(h, hu)

    return pl.pallas_call(
        body,
        grid=(grid,),
        in_specs=[
            pl.BlockSpec((K, _RU, D), lambda i: (0, i, 0)),
            pl.BlockSpec((_RU, 3 * D), lambda i: (i, 0)),
            pl.BlockSpec((D, D), lambda i: (0, 0)),
            pl.BlockSpec((D, D), lambda i: (0, 0)),
            pl.BlockSpec((D, D), lambda i: (0, 0)),
            pl.BlockSpec((1, D), lambda i: (0, 0)),
        ],
        out_specs=pl.BlockSpec((_RU, D), lambda i: (i, 0)),
        out_shape=jax.ShapeDtypeStruct(
            (E, D), jnp.float32 if last else jnp.int32),
    )(gathered, xzrh, wz_h_t, wh_h_t, ur_t, ur_b)


def kernel(fmess, bgraph, Wz_w, Wz_b, Wr_w, Ur_w, Ur_b, Wh_w, Wh_b):
    bg = bgraph.astype(jnp.int32)
    idx = bg.T.reshape(_B)
    w_x = jnp.concatenate([Wz_w[:, :D].T, Wr_w.T, Wh_w[:, :D].T], axis=1)
    b_x = jnp.concatenate([Wz_b, jnp.zeros((D,), jnp.float32), Wh_b]).reshape(1, 3 * D)
    ur_t = 0.5 * Ur_w.T
    wz_h_t = Wz_w[:, D:].T
    wh_h_t = Wh_w[:, D:].T
    ur_b = Ur_b.reshape(1, D)

    xzrh, hcat = _precompute(fmess, w_x, b_x, ur_t)
    h = None
    for d in range(2):
        gathered = _sc_gather(hcat, idx).reshape(K, E, D)
        last = d == 1
        out = _update(gathered, xzrh, wz_h_t, wh_h_t, ur_t, ur_b, last)
        if last:
            h = out
        else:
            hcat = out
    return h


# speedup vs baseline: 8.8621x; 1.0005x over previous
import functools

import jax
import jax.numpy as jnp
from jax import lax
from jax.experimental import pallas as pl
from jax.experimental.pallas import tpu as pltpu
from jax.experimental.pallas import tpu_sc as plsc

E = 160000
K = 6
D = 128

_NC = 2
_NS = 16
_NW = _NC * _NS
_CH = 120
_B = E * K
_PER_W = _B // _NW
_NCHUNK = _PER_W // _CH

_RP = 1600
_RU = 3200


def _sigmoid(x):
    return 0.5 * jnp.tanh(0.5 * x) + 0.5


def _pack(h, hu):
    hb = lax.bitcast_convert_type(h, jnp.int32) + 0x8000
    ub = lax.bitcast_convert_type(hu, jnp.int32) + 0x8000
    hi = jnp.bitwise_and(ub, jnp.int32(-65536))
    lo = lax.shift_right_logical(hb, 16)
    return jnp.bitwise_or(hi, lo)


def _unpack(w):
    h = lax.bitcast_convert_type(lax.shift_left(w, 16), jnp.float32)
    hu = lax.bitcast_convert_type(w, jnp.float32)
    return h, hu


def _sc_gather(table, idx):
    mesh = plsc.VectorSubcoreMesh(core_axis_name="c", subcore_axis_name="s")

    @functools.partial(
        pl.kernel,
        mesh=mesh,
        out_type=jax.ShapeDtypeStruct((_B, D), jnp.int32),
        scratch_types=[
            pltpu.VMEM((_PER_W,), jnp.int32),
            pltpu.VMEM((_CH, D), jnp.int32),
            pltpu.VMEM((_CH, D), jnp.int32),
            pltpu.VMEM((_CH, D), jnp.int32),
            pltpu.VMEM((_CH, D), jnp.int32),
            pltpu.SemaphoreType.DMA,
            pltpu.SemaphoreType.DMA,
            pltpu.SemaphoreType.DMA,
            pltpu.SemaphoreType.DMA,
            pltpu.SemaphoreType.DMA,
            pltpu.SemaphoreType.DMA,
            pltpu.SemaphoreType.DMA,
            pltpu.SemaphoreType.DMA,
        ],
    )
    def gather_kernel(table_hbm, idx_hbm, out_hbm, idx_all,
                      b0, b1, b2, b3, g0, g1, g2, g3, s0, s1, s2, s3):
        wid = lax.axis_index("s") * _NC + lax.axis_index("c")
        base = wid * _PER_W
        rows = (b0, b1, b2, b3)
        gsem = (g0, g1, g2, g3)
        ssem = (s0, s1, s2, s3)

        pltpu.sync_copy(idx_hbm.at[pl.ds(base, _PER_W)], idx_all)

        def fire_g(c, b):
            pltpu.async_copy(
                table_hbm.at[idx_all.at[pl.ds(c * _CH, _CH)]], rows[b], gsem[b])

        def wait_g(c, b):
            pltpu.make_async_copy(
                table_hbm.at[idx_all.at[pl.ds(c * _CH, _CH)]], rows[b],
                gsem[b]).wait()

        def fire_s(c, b):
            pltpu.async_copy(
                rows[b], out_hbm.at[pl.ds(base + c * _CH, _CH)], ssem[b])

        def wait_s(c, b):
            pltpu.make_async_copy(
                rows[b], out_hbm.at[pl.ds(base + c * _CH, _CH)], ssem[b]).wait()

        fire_g(0, 0)
        fire_g(1, 1)

        def body(i, carry):
            c = i * 4

            @pl.when(i > 0)
            def _():
                wait_s(c - 2, 2)

            fire_g(c + 2, 2)
            wait_g(c, 0)
            fire_s(c, 0)

            @pl.when(i > 0)
            def _():
                wait_s(c - 1, 3)

            fire_g(c + 3, 3)
            wait_g(c + 1, 1)
            fire_s(c + 1, 1)

            wait_s(c, 0)
            fire_g(c + 4, 0)
            wait_g(c + 2, 2)
            fire_s(c + 2, 2)

            wait_s(c + 1, 1)
            fire_g(c + 5, 1)
            wait_g(c + 3, 3)
            fire_s(c + 3, 3)
            return carry

        nbody = (_NCHUNK - 2) // 4
        lax.fori_loop(0, nbody, body, 0)
        clast = nbody * 4
        wait_g(clast, 0)
        fire_s(clast, 0)
        wait_g(clast + 1, 1)
        fire_s(clast + 1, 1)
        wait_s(clast - 2, 2)
        wait_s(clast - 1, 3)
        wait_s(clast, 0)
        wait_s(clast + 1, 1)

    return gather_kernel(table, idx)


def _precompute(fmess, w_x, b_x, ur_t):
    grid = E // _RP

    def body(x_ref, w_ref, b_ref, u_ref, xzrh_ref, hcat_ref):
        x = x_ref[...]
        xzrh = jnp.dot(x.astype(jnp.bfloat16), w_ref[...].astype(jnp.bfloat16),
                       preferred_element_type=jnp.float32) + b_ref[...]
        xzrh_ref[...] = xzrh.astype(jnp.bfloat16)
        h1 = _sigmoid(xzrh[:, :D]) * jnp.tanh(xzrh[:, 2 * D:])
        rows = lax.broadcasted_iota(jnp.int32, (_RP, 1), 0)
        keep = jnp.where((rows == 0) & (pl.program_id(0) == 0), 0.0, 1.0)
        h1 = h1 * keep
        hu1 = jnp.dot(h1.astype(jnp.bfloat16), u_ref[...].astype(jnp.bfloat16),
                      preferred_element_type=jnp.float32)
        hcat_ref[...] = _pack(h1, hu1)

    return pl.pallas_call(
        body,
        grid=(grid,),
        in_specs=[
            pl.BlockSpec((_RP, D), lambda i: (i, 0)),
            pl.BlockSpec((D, 3 * D), lambda i: (0, 0)),
            pl.BlockSpec((1, 3 * D), lambda i: (0, 0)),
            pl.BlockSpec((D, D), lambda i: (0, 0)),
        ],
        out_specs=[
            pl.BlockSpec((_RP, 3 * D), lambda i: (i, 0)),
            pl.BlockSpec((_RP, D), lambda i: (i, 0)),
        ],
        out_shape=[
            jax.ShapeDtypeStruct((E, 3 * D), jnp.bfloat16),
            jax.ShapeDtypeStruct((E, D), jnp.int32),
        ],
    )(fmess, w_x, b_x, ur_t)


def _update(gathered, xzrh, wz_h_t, wh_h_t, ur_t, ur_b, last):
    grid = E // _RU

    def body(g_ref, x_ref, wz_ref, wh_ref, u_ref, ub_ref, out_ref):
        x = x_ref[...].astype(jnp.float32)
        xz = x[:, :D]
        xr = x[:, D:2 * D]
        xh = x[:, 2 * D:]
        ub = ub_ref[...]
        xr2 = 0.5 * (xr + ub)
        sum_h = jnp.zeros((_RU, D), jnp.float32)
        sum_t = jnp.zeros((_RU, D), jnp.float32)
        for k in range(K):
            hk, huk2 = _unpack(g_ref[k, :, :])
            sum_h = sum_h + hk
            tk = jnp.tanh(xr2 + huk2)
            sum_t = sum_t + tk * hk
        sum_g = 0.5 * (sum_h + sum_t)
        z = _sigmoid(xz + jnp.dot(sum_h.astype(jnp.bfloat16), wz_ref[...].astype(jnp.bfloat16), preferred_element_type=jnp.float32))
        pre = jnp.tanh(xh + jnp.dot(sum_g.astype(jnp.bfloat16), wh_ref[...].astype(jnp.bfloat16), preferred_element_type=jnp.float32))
        h = (1.0 - z) * sum_h + z * pre
        rows = lax.broadcasted_iota(jnp.int32, (_RU, 1), 0)
        keep = jnp.where((rows == 0) & (pl.program_id(0) == 0), 0.0, 1.0)
        h = h * keep
        if last:
            out_ref[...] = h
        else:
            hu = jnp.dot(h.astype(jnp.bfloat16), u_ref[...].astype(jnp.bfloat16),
                         preferred_element_type=jnp.float32)
            out_ref[...] = _pack(h, hu)

    return pl.pallas_call(
        body,
        grid=(grid,),
        in_specs=[
            pl.BlockSpec((K, _RU, D), lambda i: (0, i, 0)),
            pl.BlockSpec((_RU, 3 * D), lambda i: (i, 0)),
            pl.BlockSpec((D, D), lambda i: (0, 0)),
            pl.BlockSpec((D, D), lambda i: (0, 0)),
            pl.BlockSpec((D, D), lambda i: (0, 0)),
            pl.BlockSpec((1, D), lambda i: (0, 0)),
        ],
        out_specs=pl.BlockSpec((_RU, D), lambda i: (i, 0)),
        out_shape=jax.ShapeDtypeStruct(
            (E, D), jnp.float32 if last else jnp.int32),
    )(gathered, xzrh, wz_h_t, wh_h_t, ur_t, ur_b)


def kernel(fmess, bgraph, Wz_w, Wz_b, Wr_w, Ur_w, Ur_b, Wh_w, Wh_b):
    bg = bgraph.astype(jnp.int32)
    idx = bg.T.reshape(_B)
    w_x = jnp.concatenate([Wz_w[:, :D].T, Wr_w.T, Wh_w[:, :D].T], axis=1)
    b_x = jnp.concatenate([Wz_b, jnp.zeros((D,), jnp.float32), Wh_b]).reshape(1, 3 * D)
    ur_t = 0.5 * Ur_w.T
    wz_h_t = Wz_w[:, D:].T
    wh_h_t = Wh_w[:, D:].T
    ur_b = Ur_b.reshape(1, D)

    xzrh, hcat = _precompute(fmess, w_x, b_x, ur_t)
    h = None
    for d in range(2):
        gathered = _sc_gather(hcat, idx).reshape(K, E, D)
        last = d == 1
        out = _update(gathered, xzrh, wz_h_t, wh_h_t, ur_t, ur_b, last)
        if last:
            h = out
        else:
            hcat = out
    return h


# speedup vs baseline: 8.8622x; 1.0000x over previous
import functools

import jax
import jax.numpy as jnp
from jax import lax
from jax.experimental import pallas as pl
from jax.experimental.pallas import tpu as pltpu
from jax.experimental.pallas import tpu_sc as plsc

E = 160000
K = 6
D = 128

_NC = 2
_NS = 16
_NW = _NC * _NS
_CH = 120
_B = E * K
_PER_W = _B // _NW
_NCHUNK = _PER_W // _CH

_RP = 1600
_RU = 3200


def _sigmoid(x):
    return 0.5 * jnp.tanh(0.5 * x) + 0.5


def _pack(h, hu):
    hb = lax.bitcast_convert_type(h, jnp.int32) + 0x8000
    ub = lax.bitcast_convert_type(hu, jnp.int32) + 0x8000
    hi = jnp.bitwise_and(ub, jnp.int32(-65536))
    lo = lax.shift_right_logical(hb, 16)
    return jnp.bitwise_or(hi, lo)


def _unpack(w):
    h = lax.bitcast_convert_type(lax.shift_left(w, 16), jnp.float32)
    hu = lax.bitcast_convert_type(w, jnp.float32)
    return h, hu


def _sc_gather(table, idx):
    mesh = plsc.VectorSubcoreMesh(core_axis_name="c", subcore_axis_name="s")

    @functools.partial(
        pl.kernel,
        mesh=mesh,
        out_type=jax.ShapeDtypeStruct((_B, D), jnp.int32),
        scratch_types=[
            pltpu.VMEM((_PER_W,), jnp.int32),
            pltpu.VMEM((_CH, D), jnp.int32),
            pltpu.VMEM((_CH, D), jnp.int32),
            pltpu.VMEM((_CH, D), jnp.int32),
            pltpu.VMEM((_CH, D), jnp.int32),
            pltpu.SemaphoreType.DMA,
            pltpu.SemaphoreType.DMA,
            pltpu.SemaphoreType.DMA,
            pltpu.SemaphoreType.DMA,
            pltpu.SemaphoreType.DMA,
            pltpu.SemaphoreType.DMA,
            pltpu.SemaphoreType.DMA,
            pltpu.SemaphoreType.DMA,
        ],
    )
    def gather_kernel(table_hbm, idx_hbm, out_hbm, idx_all,
                      b0, b1, b2, b3, g0, g1, g2, g3, s0, s1, s2, s3):
        wid = lax.axis_index("s") * _NC + lax.axis_index("c")
        base = wid * _PER_W
        rows = (b0, b1, b2, b3)
        gsem = (g0, g1, g2, g3)
        ssem = (s0, s1, s2, s3)

        pltpu.sync_copy(idx_hbm.at[pl.ds(base, _PER_W)], idx_all)

        def fire_g(c, b):
            pltpu.async_copy(
                table_hbm.at[idx_all.at[pl.ds(c * _CH, _CH)]], rows[b], gsem[b])

        def wait_g(c, b):
            pltpu.make_async_copy(
                table_hbm.at[idx_all.at[pl.ds(c * _CH, _CH)]], rows[b],
                gsem[b]).wait()

        def fire_s(c, b):
            pltpu.async_copy(
                rows[b], out_hbm.at[pl.ds(base + c * _CH, _CH)], ssem[b])

        def wait_s(c, b):
            pltpu.make_async_copy(
                rows[b], out_hbm.at[pl.ds(base + c * _CH, _CH)], ssem[b]).wait()

        fire_g(0, 0)
        fire_g(1, 1)

        def body(i, carry):
            c = i * 4

            @pl.when(i > 0)
            def _():
                wait_s(c - 2, 2)

            fire_g(c + 2, 2)
            wait_g(c, 0)
            fire_s(c, 0)

            @pl.when(i > 0)
            def _():
                wait_s(c - 1, 3)

            fire_g(c + 3, 3)
            wait_g(c + 1, 1)
            fire_s(c + 1, 1)

            wait_s(c, 0)
            fire_g(c + 4, 0)
            wait_g(c + 2, 2)
            fire_s(c + 2, 2)

            wait_s(c + 1, 1)
            fire_g(c + 5, 1)
            wait_g(c + 3, 3)
            fire_s(c + 3, 3)
            return carry

        nbody = (_NCHUNK - 2) // 4
        lax.fori_loop(0, nbody, body, 0)
        clast = nbody * 4
        wait_g(clast, 0)
        fire_s(clast, 0)
        wait_g(clast + 1, 1)
        fire_s(clast + 1, 1)
        wait_s(clast - 2, 2)
        wait_s(clast - 1, 3)
        wait_s(clast, 0)
        wait_s(clast + 1, 1)

    return gather_kernel(table, idx)


def _precompute(fmess, w_x, b_x, ur_t):
    grid = E // _RP

    def body(x_ref, w_ref, b_ref, u_ref, xzrh_ref, hcat_ref):
        x = x_ref[...]
        xzrh = jnp.dot(x, w_ref[...], preferred_element_type=jnp.float32) + b_ref[...]
        xzrh_ref[...] = xzrh.astype(jnp.bfloat16)
        h1 = _sigmoid(xzrh[:, :D]) * jnp.tanh(xzrh[:, 2 * D:])
        rows = lax.broadcasted_iota(jnp.int32, (_RP, 1), 0)
        keep = jnp.where((rows == 0) & (pl.program_id(0) == 0), 0.0, 1.0)
        h1 = h1 * keep
        hu1 = jnp.dot(h1.astype(jnp.bfloat16), u_ref[...].astype(jnp.bfloat16),
                      preferred_element_type=jnp.float32)
        hcat_ref[...] = _pack(h1, hu1)

    return pl.pallas_call(
        body,
        grid=(grid,),
        in_specs=[
            pl.BlockSpec((_RP, D), lambda i: (i, 0)),
            pl.BlockSpec((D, 3 * D), lambda i: (0, 0)),
            pl.BlockSpec((1, 3 * D), lambda i: (0, 0)),
            pl.BlockSpec((D, D), lambda i: (0, 0)),
        ],
        out_specs=[
            pl.BlockSpec((_RP, 3 * D), lambda i: (i, 0)),
            pl.BlockSpec((_RP, D), lambda i: (i, 0)),
        ],
        out_shape=[
            jax.ShapeDtypeStruct((E, 3 * D), jnp.bfloat16),
            jax.ShapeDtypeStruct((E, D), jnp.int32),
        ],
    )(fmess, w_x, b_x, ur_t)


def _update(gathered, xzrh, wz_h_t, wh_h_t, ur_t, ur_b, last):
    grid = E // _RU

    def body(g_ref, x_ref, wz_ref, wh_ref, u_ref, ub_ref, out_ref):
        x = x_ref[...].astype(jnp.float32)
        xz = x[:, :D]
        xr = x[:, D:2 * D]
        xh = x[:, 2 * D:]
        ub = ub_ref[...]
        xr2 = 0.5 * (xr + ub)
        sum_h = jnp.zeros((_RU, D), jnp.float32)
        sum_t = jnp.zeros((_RU, D), jnp.float32)
        for k in range(K):
            hk, huk2 = _unpack(g_ref[k, :, :])
            sum_h = sum_h + hk
            tk = jnp.tanh(xr2 + huk2)
            sum_t = sum_t + tk * hk
        sum_g = 0.5 * (sum_h + sum_t)
        z = _sigmoid(xz + jnp.dot(sum_h.astype(jnp.bfloat16), wz_ref[...].astype(jnp.bfloat16), preferred_element_type=jnp.float32))
        pre = jnp.tanh(xh + jnp.dot(sum_g.astype(jnp.bfloat16), wh_ref[...].astype(jnp.bfloat16), preferred_element_type=jnp.float32))
        h = (1.0 - z) * sum_h + z * pre
        rows = lax.broadcasted_iota(jnp.int32, (_RU, 1), 0)
        keep = jnp.where((rows == 0) & (pl.program_id(0) == 0), 0.0, 1.0)
        h = h * keep
        if last:
            out_ref[...] = h
        else:
            hu = jnp.dot(h.astype(jnp.bfloat16), u_ref[...].astype(jnp.bfloat16),
                         preferred_element_type=jnp.float32)
            out_ref[...] = _pack(h, hu)

    return pl.pallas_call(
        body,
        grid=(grid,),
        in_specs=[
            pl.BlockSpec((K, _RU, D), lambda i: (0, i, 0)),
            pl.BlockSpec((_RU, 3 * D), lambda i: (i, 0)),
            pl.BlockSpec((D, D), lambda i: (0, 0)),
            pl.BlockSpec((D, D), lambda i: (0, 0)),
            pl.BlockSpec((D, D), lambda i: (0, 0)),
            pl.BlockSpec((1, D), lambda i: (0, 0)),
        ],
        out_specs=pl.BlockSpec((_RU, D), lambda i: (i, 0)),
        out_shape=jax.ShapeDtypeStruct(
            (E, D), jnp.float32 if last else jnp.int32),
    )(gathered, xzrh, wz_h_t, wh_h_t, ur_t, ur_b)


def kernel(fmess, bgraph, Wz_w, Wz_b, Wr_w, Ur_w, Ur_b, Wh_w, Wh_b):
    bg = bgraph.astype(jnp.int32)
    idx = bg.T.reshape(_B)
    w_x = jnp.concatenate([Wz_w[:, :D].T, Wr_w.T, Wh_w[:, :D].T], axis=1)
    b_x = jnp.concatenate([Wz_b, jnp.zeros((D,), jnp.float32), Wh_b]).reshape(1, 3 * D)
    ur_t = 0.5 * Ur_w.T
    wz_h_t = Wz_w[:, D:].T
    wh_h_t = Wh_w[:, D:].T
    ur_b = Ur_b.reshape(1, D)

    xzrh, hcat = _precompute(fmess, w_x, b_x, ur_t)
    h = None
    for d in range(2):
        gathered = _sc_gather(hcat, idx).reshape(K, E, D)
        last = d == 1
        out = _update(gathered, xzrh, wz_h_t, wh_h_t, ur_t, ur_b, last)
        if last:
            h = out
        else:
            hcat = out
    return h
